# Initial kernel scaffold; baseline (speedup 1.0000x reference)
import jax
import jax.numpy as jnp
from jax.experimental import pallas as pl


def kernel(source_vertices, source_faces, target_vertices, target_faces, face_probs):
    raise NotImplementedError("write your pallas kernel here")


# speedup vs baseline: 11.8808x; 11.8808x over previous
import functools

import jax
import jax.numpy as jnp
from jax import lax
from jax.experimental import pallas as pl
from jax.experimental.pallas import tpu as pltpu

NPF = 4
KNN = 5
F_SRC = 2048
N_PTS = F_SRC * NPF
N_TGT = 4096
BLK_P = 512
BLK_F = BLK_P // NPF
GRID = N_PTS // BLK_P
TGT_CHUNK = 1024


def _dense_body(px_ref, sb_ref, pc_ref, py_ref, sx_ref, ty_ref, pr_ref, out_ref):
    b = pl.program_id(0)
    px = px_ref[...]
    sb = sb_ref[...]
    pc = pc_ref[...]
    py = py_ref[...]
    pr = pr_ref[...]

    d = jnp.zeros((BLK_P, F_SRC), jnp.float32)
    for c in range(3):
        dx = px[:, c:c + 1] - sx_ref[c:c + 1, :]
        d = d + dx * dx

    ii = lax.broadcasted_iota(jnp.int32, (BLK_P, F_SRC), 1)
    face_col = (lax.broadcasted_iota(jnp.int32, (BLK_P, 1), 0) + b * BLK_P) // NPF
    tot = jnp.zeros((BLK_P, 1), jnp.float32)
    s_self = jnp.zeros((BLK_P, 1), jnp.float32)
    has_self = jnp.zeros((BLK_P, 1), jnp.bool_)
    last_pd = jnp.zeros((BLK_P, 1), jnp.float32)
    inf = jnp.float32(jnp.inf)
    for i in range(KNN + 1):
        m = jnp.min(d, axis=1, keepdims=True)
        eq = d == m
        idx = jnp.min(jnp.where(eq, ii, F_SRC), axis=1, keepdims=True)
        oh = ii == idx
        p = jnp.sum(jnp.where(oh, pr, 0.0), axis=1, keepdims=True)
        pd = p * m
        tot = tot + pd
        selfhit = idx == face_col
        s_self = s_self + jnp.where(selfhit, pd, 0.0)
        has_self = has_self | selfhit
        if i == KNN:
            last_pd = pd
        else:
            d = jnp.where(oh, inf, d)
    mean_term = jnp.where(has_self, tot - s_self, tot - last_pd) * (1.0 / KNN)

    mt = jnp.full((BLK_P, 1), inf, jnp.float32)
    fmin = jnp.full((BLK_F, 1), inf, jnp.float32)
    for t in range(N_TGT // TGT_CHUNK):
        dt = jnp.zeros((BLK_P, TGT_CHUNK), jnp.float32)
        df = jnp.zeros((BLK_F, TGT_CHUNK), jnp.float32)
        for c in range(3):
            tyc = ty_ref[c:c + 1, t * TGT_CHUNK:(t + 1) * TGT_CHUNK]
            dx = px[:, c:c + 1] - tyc
            dt = dt + dx * dx
            dxf = sb[:, c:c + 1] - tyc
            df = df + dxf * dxf
        mt = jnp.minimum(mt, jnp.min(dt, axis=1, keepdims=True))
        fmin = jnp.minimum(fmin, jnp.min(df, axis=1, keepdims=True))

    rev = jnp.sum(py * mt + (1.0 - py) * mean_term)
    fwd = jnp.sum(pc * fmin)

    @pl.when(b == 0)
    def _():
        out_ref[...] = jnp.zeros((1, 1), jnp.float32)

    out_ref[...] += (rev + fwd).reshape(1, 1)


def _dense_loss(px, sb_col, pc_col, py_col, sx_row, ty_row, pr_row):
    return pl.pallas_call(
        _dense_body,
        grid=(GRID,),
        in_specs=[
            pl.BlockSpec((BLK_P, 128), lambda b: (b, 0)),
            pl.BlockSpec((BLK_F, 128), lambda b: (b, 0)),
            pl.BlockSpec((BLK_F, 1), lambda b: (b, 0)),
            pl.BlockSpec((BLK_P, 1), lambda b: (b, 0)),
            pl.BlockSpec((8, F_SRC), lambda b: (0, 0)),
            pl.BlockSpec((8, N_TGT), lambda b: (0, 0)),
            pl.BlockSpec((1, F_SRC), lambda b: (0, 0)),
        ],
        out_specs=pl.BlockSpec((1, 1), lambda b: (0, 0)),
        out_shape=jax.ShapeDtypeStruct((1, 1), jnp.float32),
    )(px, sb_col, pc_col, py_col, sx_row, ty_row, pr_row)


def _sample_weights():
    rk = jax.random.key(42)
    r1 = jnp.sqrt(jax.random.uniform(jax.random.fold_in(rk, 0), (F_SRC, NPF), dtype=jnp.float32))
    r2 = jax.random.uniform(jax.random.fold_in(rk, 1), (F_SRC, NPF), dtype=jnp.float32)
    w1 = 1.0 - r1
    w2 = r1 * (1.0 - r2)
    w3 = r1 * r2
    return w1, w2, w3


def kernel(source_vertices, source_faces, target_vertices, target_faces, face_probs):
    sv = source_vertices[0]
    tv = target_vertices[0]
    sf = source_faces.astype(jnp.int32)
    tf = target_faces.astype(jnp.int32)
    w1, w2, w3 = _sample_weights()

    v1 = sv[sf[:, 0]]
    v2 = sv[sf[:, 1]]
    v3 = sv[sf[:, 2]]
    src_bary = (v1 + v2 + v3) / 3.0
    points = (w1[..., None] * v1[:, None, :] + w2[..., None] * v2[:, None, :]
              + w3[..., None] * v3[:, None, :]).reshape(N_PTS, 3)
    tgt_bary = tv[tf.T].mean(axis=1)

    px = jnp.zeros((N_PTS, 128), jnp.float32).at[:, 0:3].set(points)
    sb_col = jnp.zeros((F_SRC, 128), jnp.float32).at[:, 0:3].set(src_bary)
    sx_row = jnp.zeros((8, F_SRC), jnp.float32).at[0:3, :].set(src_bary.T)
    ty_row = jnp.zeros((8, N_TGT), jnp.float32).at[0:3, :].set(tgt_bary.T)
    pr_row = face_probs[None, :]
    pc_col = face_probs[:, None]
    py_col = jnp.repeat(face_probs, NPF)[:, None]

    loss = _dense_loss(px, sb_col, pc_col, py_col, sx_row, ty_row, pr_row)
    return loss[0, 0]


# speedup vs baseline: 12.3978x; 1.0435x over previous
import functools

import jax
import jax.numpy as jnp
from jax import lax
from jax.experimental import pallas as pl
from jax.experimental.pallas import tpu as pltpu
from jax.experimental.pallas import tpu_sc as plsc

SC_CORES = 2
SC_SUBCORES = 16
SC_WORKERS = SC_CORES * SC_SUBCORES
SC_LANES = 16

NPF = 4
KNN = 5
F_SRC = 2048
N_PTS = F_SRC * NPF
N_TGT = 4096
BLK_P = 512
BLK_F = BLK_P // NPF
GRID = N_PTS // BLK_P
TGT_CHUNK = 1024


def _dense_body(px_ref, sb_ref, pc_ref, py_ref, sx_ref, ty_ref, pr_ref, out_ref):
    b = pl.program_id(0)
    px = px_ref[...]
    sb = sb_ref[...]
    pc = pc_ref[...]
    py = py_ref[...]
    pr = pr_ref[...]

    pxb = px.astype(jnp.bfloat16).astype(jnp.float32)
    sbb = sb.astype(jnp.bfloat16).astype(jnp.float32)

    px2 = jnp.zeros((BLK_P, 1), jnp.float32)
    sx2 = jnp.zeros((1, F_SRC), jnp.float32)
    cross = jnp.zeros((BLK_P, F_SRC), jnp.float32)
    for c in range(3):
        px2 = px2 + px[:, c:c + 1] * px[:, c:c + 1]
        sxc = sx_ref[c:c + 1, :]
        sx2 = sx2 + sxc * sxc
        cross = cross + pxb[:, c:c + 1] * sxc.astype(jnp.bfloat16).astype(jnp.float32)
    d = jnp.maximum(px2 + sx2 - 2.0 * cross, 0.0)

    ii = lax.broadcasted_iota(jnp.int32, (BLK_P, F_SRC), 1)
    face_col = (lax.broadcasted_iota(jnp.int32, (BLK_P, 1), 0) + b * BLK_P) // NPF
    tot = jnp.zeros((BLK_P, 1), jnp.float32)
    s_self = jnp.zeros((BLK_P, 1), jnp.float32)
    has_self = jnp.zeros((BLK_P, 1), jnp.bool_)
    last_pd = jnp.zeros((BLK_P, 1), jnp.float32)
    inf = jnp.float32(jnp.inf)
    for i in range(KNN + 1):
        m = jnp.min(d, axis=1, keepdims=True)
        eq = d == m
        idx = jnp.min(jnp.where(eq, ii, F_SRC), axis=1, keepdims=True)
        oh = ii == idx
        p = jnp.sum(jnp.where(oh, pr, 0.0), axis=1, keepdims=True)
        pd = p * m
        tot = tot + pd
        selfhit = idx == face_col
        s_self = s_self + jnp.where(selfhit, pd, 0.0)
        has_self = has_self | selfhit
        if i == KNN:
            last_pd = pd
        else:
            d = jnp.where(oh, inf, d)
    mean_term = jnp.where(has_self, tot - s_self, tot - last_pd) * (1.0 / KNN)

    sb2 = jnp.zeros((BLK_F, 1), jnp.float32)
    for c in range(3):
        sb2 = sb2 + sb[:, c:c + 1] * sb[:, c:c + 1]
    mt = jnp.full((BLK_P, 1), inf, jnp.float32)
    fmin = jnp.full((BLK_F, 1), inf, jnp.float32)
    for t in range(N_TGT // TGT_CHUNK):
        ty2 = jnp.zeros((1, TGT_CHUNK), jnp.float32)
        ct = jnp.zeros((BLK_P, TGT_CHUNK), jnp.float32)
        cf = jnp.zeros((BLK_F, TGT_CHUNK), jnp.float32)
        for c in range(3):
            tyc = ty_ref[c:c + 1, t * TGT_CHUNK:(t + 1) * TGT_CHUNK]
            tyb = tyc.astype(jnp.bfloat16).astype(jnp.float32)
            ty2 = ty2 + tyc * tyc
            ct = ct + pxb[:, c:c + 1] * tyb
            cf = cf + sbb[:, c:c + 1] * tyb
        dt = jnp.maximum(px2 + ty2 - 2.0 * ct, 0.0)
        df = jnp.maximum(sb2 + ty2 - 2.0 * cf, 0.0)
        mt = jnp.minimum(mt, jnp.min(dt, axis=1, keepdims=True))
        fmin = jnp.minimum(fmin, jnp.min(df, axis=1, keepdims=True))

    rev = jnp.sum(py * mt + (1.0 - py) * mean_term)
    fwd = jnp.sum(pc * fmin)

    @pl.when(b == 0)
    def _():
        out_ref[...] = jnp.zeros((1, 1), jnp.float32)

    out_ref[...] += (rev + fwd).reshape(1, 1)


def _dense_loss(px, sb_col, pc_col, py_col, sx_row, ty_row, pr_row):
    return pl.pallas_call(
        _dense_body,
        grid=(GRID,),
        in_specs=[
            pl.BlockSpec((BLK_P, 128), lambda b: (b, 0)),
            pl.BlockSpec((BLK_F, 128), lambda b: (b, 0)),
            pl.BlockSpec((BLK_F, 1), lambda b: (b, 0)),
            pl.BlockSpec((BLK_P, 1), lambda b: (b, 0)),
            pl.BlockSpec((8, F_SRC), lambda b: (0, 0)),
            pl.BlockSpec((8, N_TGT), lambda b: (0, 0)),
            pl.BlockSpec((1, F_SRC), lambda b: (0, 0)),
        ],
        out_specs=pl.BlockSpec((1, 1), lambda b: (0, 0)),
        out_shape=jax.ShapeDtypeStruct((1, 1), jnp.float32),
    )(px, sb_col, pc_col, py_col, sx_row, ty_row, pr_row)


def _sc_body(sv_h, tv_h, sf_h, tf_h, w_h, sb_h, pt_h, tb_h,
             sv_v, tv_v, sf_v, tf_v, w_v, bb_v, pp_v, tb_v):
    wid = lax.axis_index("s") * SC_CORES + lax.axis_index("c")
    pltpu.sync_copy(sv_h, sv_v)
    pltpu.sync_copy(tv_h, tv_v)
    pltpu.sync_copy(sf_h, sf_v)
    pltpu.sync_copy(tf_h, tf_v)
    pltpu.sync_copy(w_h, w_v)

    f_per_w = F_SRC // SC_WORKERS
    t_per_w = N_TGT // SC_WORKERS
    base = wid * f_per_w
    third = jnp.float32(1.0 / 3.0)
    for u in range(f_per_w // SC_LANES):
        o = base + u * SC_LANES
        f1 = sf_v[0, pl.ds(o, SC_LANES)]
        f2 = sf_v[1, pl.ds(o, SC_LANES)]
        f3 = sf_v[2, pl.ds(o, SC_LANES)]
        for c in range(3):
            cc = jnp.full((SC_LANES,), c, jnp.int32)
            v1 = plsc.load_gather(sv_v, [f1 * 3 + cc])
            v2 = plsc.load_gather(sv_v, [f2 * 3 + cc])
            v3 = plsc.load_gather(sv_v, [f3 * 3 + cc])
            bb_v[c, pl.ds(u * SC_LANES, SC_LANES)] = (v1 + v2 + v3) * third
            for j in range(NPF):
                w1c = w_v[0, j, pl.ds(o, SC_LANES)]
                w2c = w_v[1, j, pl.ds(o, SC_LANES)]
                w3c = w_v[2, j, pl.ds(o, SC_LANES)]
                pp_v[c, j, pl.ds(u * SC_LANES, SC_LANES)] = (
                    w1c * v1 + w2c * v2 + w3c * v3)
    for c in range(3):
        pltpu.sync_copy(bb_v.at[c], sb_h.at[pl.ds(c * F_SRC + base, f_per_w)])
        for j in range(NPF):
            pltpu.sync_copy(pp_v.at[c, j],
                            pt_h.at[pl.ds((c * NPF + j) * F_SRC + base, f_per_w)])

    tbase = wid * t_per_w
    for u in range(t_per_w // SC_LANES):
        o = tbase + u * SC_LANES
        t1 = tf_v[0, pl.ds(o, SC_LANES)]
        t2 = tf_v[1, pl.ds(o, SC_LANES)]
        t3 = tf_v[2, pl.ds(o, SC_LANES)]
        for c in range(3):
            cc = jnp.full((SC_LANES,), c, jnp.int32)
            g = (plsc.load_gather(tv_v, [t1 * 3 + cc])
                 + plsc.load_gather(tv_v, [t2 * 3 + cc])
                 + plsc.load_gather(tv_v, [t3 * 3 + cc]))
            tb_v[c, pl.ds(u * SC_LANES, SC_LANES)] = g * third
    for c in range(3):
        pltpu.sync_copy(tb_v.at[c], tb_h.at[pl.ds(c * N_TGT + tbase, t_per_w)])


def _sc_gather(svr, tvr, sft, tft, wst):
    n_sv = svr.shape[0]
    n_tv = tvr.shape[0]
    fn = pl.kernel(
        _sc_body,
        out_type=[
            jax.ShapeDtypeStruct((3 * F_SRC,), jnp.float32),
            jax.ShapeDtypeStruct((3 * NPF * F_SRC,), jnp.float32),
            jax.ShapeDtypeStruct((3 * N_TGT,), jnp.float32),
        ],
        mesh=plsc.VectorSubcoreMesh(core_axis_name="c", subcore_axis_name="s"),
        compiler_params=pltpu.CompilerParams(needs_layout_passes=False),
        scratch_types=[
            pltpu.VMEM((n_sv * 3,), jnp.float32),
            pltpu.VMEM((n_tv * 3,), jnp.float32),
            pltpu.VMEM((3, F_SRC), jnp.int32),
            pltpu.VMEM((3, N_TGT), jnp.int32),
            pltpu.VMEM((3, NPF, F_SRC), jnp.float32),
            pltpu.VMEM((3, F_SRC // SC_WORKERS), jnp.float32),
            pltpu.VMEM((3, NPF, F_SRC // SC_WORKERS), jnp.float32),
            pltpu.VMEM((3, N_TGT // SC_WORKERS), jnp.float32),
        ],
    )
    sb_f, pt_f, tb_f = fn(svr.reshape(-1), tvr.reshape(-1), sft, tft, wst)
    return (sb_f.reshape(3, F_SRC), pt_f.reshape(3, NPF, F_SRC),
            tb_f.reshape(3, N_TGT))


def _sample_weights():
    rk = jax.random.key(42)
    r1 = jnp.sqrt(jax.random.uniform(jax.random.fold_in(rk, 0), (F_SRC, NPF), dtype=jnp.float32))
    r2 = jax.random.uniform(jax.random.fold_in(rk, 1), (F_SRC, NPF), dtype=jnp.float32)
    w1 = 1.0 - r1
    w2 = r1 * (1.0 - r2)
    w3 = r1 * r2
    return w1, w2, w3


def kernel(source_vertices, source_faces, target_vertices, target_faces, face_probs):
    sv = source_vertices[0]
    tv = target_vertices[0]
    sf = source_faces.astype(jnp.int32)
    tf = target_faces.astype(jnp.int32)
    w1, w2, w3 = _sample_weights()

    wst = jnp.stack([w1.T, w2.T, w3.T])
    sb_t, pt_t, tb_t = _sc_gather(sv, tv, sf.T, tf, wst)
    points = jnp.transpose(pt_t, (2, 1, 0)).reshape(N_PTS, 3)

    px = jnp.zeros((N_PTS, 128), jnp.float32).at[:, 0:3].set(points)
    sb_col = jnp.zeros((F_SRC, 128), jnp.float32).at[:, 0:3].set(sb_t.T)
    sx_row = jnp.zeros((8, F_SRC), jnp.float32).at[0:3, :].set(sb_t)
    ty_row = jnp.zeros((8, N_TGT), jnp.float32).at[0:3, :].set(tb_t)
    pr_row = face_probs[None, :]
    pc_col = face_probs[:, None]
    py_col = jnp.repeat(face_probs, NPF)[:, None]

    loss = _dense_loss(px, sb_col, pc_col, py_col, sx_row, ty_row, pr_row)
    return loss[0, 0]


# speedup vs baseline: 15.4429x; 1.2456x over previous
import jax
import jax.numpy as jnp
from jax import lax
from jax.experimental import pallas as pl
from jax.experimental.pallas import tpu as pltpu
from jax.experimental.pallas import tpu_sc as plsc

SC_CORES = 2
SC_SUBCORES = 16
SC_WORKERS = SC_CORES * SC_SUBCORES
SC_LANES = 16

NPF = 4
KNN = 5
F_SRC = 2048
N_PTS = F_SRC * NPF
N_TGT = 4096
BLK_P = 512
BLK_F = BLK_P // NPF
GRID = N_PTS // BLK_P

IDX_BITS = 11
IDX_MASK = (1 << IDX_BITS) - 1


def _dense_body(px_ref, pxb_ref, sb_ref, sbb_ref, pc_ref, py_ref,
                sx_ref, sxbt_ref, ty_ref, tybt_ref, pr_ref, out_ref):
    b = pl.program_id(0)
    px = px_ref[...]
    sb = sb_ref[...]
    pc = pc_ref[...]
    py = py_ref[...]
    pr = pr_ref[...]

    px2 = jnp.zeros((BLK_P, 1), jnp.float32)
    sx2 = jnp.zeros((1, F_SRC), jnp.float32)
    sb2 = jnp.zeros((BLK_F, 1), jnp.float32)
    ty2 = jnp.zeros((1, N_TGT), jnp.float32)
    for c in range(3):
        px2 = px2 + px[:, c:c + 1] * px[:, c:c + 1]
        sxc = sx_ref[c:c + 1, :]
        sx2 = sx2 + sxc * sxc
        sb2 = sb2 + sb[:, c:c + 1] * sb[:, c:c + 1]
        tyc = ty_ref[c:c + 1, :]
        ty2 = ty2 + tyc * tyc

    cross = jnp.dot(pxb_ref[...], sxbt_ref[...],
                    preferred_element_type=jnp.float32)
    d = jnp.maximum(px2 + sx2 - 2.0 * cross, 0.0)

    ii = lax.broadcasted_iota(jnp.int32, (BLK_P, F_SRC), 1)
    key = (d.view(jnp.int32) & ~IDX_MASK) | ii
    face_col = (lax.broadcasted_iota(jnp.int32, (BLK_P, 1), 0) + b * BLK_P) // NPF
    tot = jnp.zeros((BLK_P, 1), jnp.float32)
    s_self = jnp.zeros((BLK_P, 1), jnp.float32)
    has_self = jnp.zeros((BLK_P, 1), jnp.bool_)
    last_pd = jnp.zeros((BLK_P, 1), jnp.float32)
    imax = jnp.int32(0x7FFFFFFF)
    for i in range(KNN + 1):
        kmin = jnp.min(key, axis=1, keepdims=True)
        oh = key == kmin
        p = jnp.sum(jnp.where(oh, pr, 0.0), axis=1, keepdims=True)
        dq = (kmin & ~IDX_MASK).view(jnp.float32)
        idx = kmin & IDX_MASK
        pd = p * dq
        tot = tot + pd
        selfhit = idx == face_col
        s_self = s_self + jnp.where(selfhit, pd, 0.0)
        has_self = has_self | selfhit
        if i == KNN:
            last_pd = pd
        else:
            key = jnp.where(oh, imax, key)
    mean_term = jnp.where(has_self, tot - s_self, tot - last_pd) * (1.0 / KNN)

    ct = jnp.dot(pxb_ref[...], tybt_ref[...],
                 preferred_element_type=jnp.float32)
    dt = jnp.maximum(px2 + ty2 - 2.0 * ct, 0.0)
    mt = jnp.min(dt, axis=1, keepdims=True)
    cf = jnp.dot(sbb_ref[...], tybt_ref[...],
                 preferred_element_type=jnp.float32)
    df = jnp.maximum(sb2 + ty2 - 2.0 * cf, 0.0)
    fmin = jnp.min(df, axis=1, keepdims=True)

    rev = jnp.sum(py * mt + (1.0 - py) * mean_term)
    fwd = jnp.sum(pc * fmin)

    @pl.when(b == 0)
    def _():
        out_ref[...] = jnp.zeros((1, 1), jnp.float32)

    out_ref[...] += (rev + fwd).reshape(1, 1)


def _dense_loss(px, pxb, sb_col, sbb, pc_col, py_col, sx_row, sxbt, ty_row,
                tybt, pr_row):
    return pl.pallas_call(
        _dense_body,
        grid=(GRID,),
        in_specs=[
            pl.BlockSpec((BLK_P, 128), lambda b: (b, 0)),
            pl.BlockSpec((BLK_P, 128), lambda b: (b, 0)),
            pl.BlockSpec((BLK_F, 128), lambda b: (b, 0)),
            pl.BlockSpec((BLK_F, 128), lambda b: (b, 0)),
            pl.BlockSpec((BLK_F, 1), lambda b: (b, 0)),
            pl.BlockSpec((BLK_P, 1), lambda b: (b, 0)),
            pl.BlockSpec((8, F_SRC), lambda b: (0, 0)),
            pl.BlockSpec((128, F_SRC), lambda b: (0, 0)),
            pl.BlockSpec((8, N_TGT), lambda b: (0, 0)),
            pl.BlockSpec((128, N_TGT), lambda b: (0, 0)),
            pl.BlockSpec((1, F_SRC), lambda b: (0, 0)),
        ],
        out_specs=pl.BlockSpec((1, 1), lambda b: (0, 0)),
        out_shape=jax.ShapeDtypeStruct((1, 1), jnp.float32),
    )(px, pxb, sb_col, sbb, pc_col, py_col, sx_row, sxbt, ty_row, tybt, pr_row)


def _sc_body(sv_h, tv_h, sf_h, tf_h, w_h, sb_h, pt_h, tb_h,
             sv_v, tv_v, sf_v, tf_v, w_v, bb_v, pp_v, tb_v):
    wid = lax.axis_index("s") * SC_CORES + lax.axis_index("c")
    pltpu.sync_copy(sv_h, sv_v)
    pltpu.sync_copy(tv_h, tv_v)
    pltpu.sync_copy(sf_h, sf_v)
    pltpu.sync_copy(tf_h, tf_v)
    pltpu.sync_copy(w_h, w_v)

    f_per_w = F_SRC // SC_WORKERS
    t_per_w = N_TGT // SC_WORKERS
    base = wid * f_per_w
    third = jnp.float32(1.0 / 3.0)
    for u in range(f_per_w // SC_LANES):
        o = base + u * SC_LANES
        f1 = sf_v[0, pl.ds(o, SC_LANES)]
        f2 = sf_v[1, pl.ds(o, SC_LANES)]
        f3 = sf_v[2, pl.ds(o, SC_LANES)]
        for c in range(3):
            cc = jnp.full((SC_LANES,), c, jnp.int32)
            v1 = plsc.load_gather(sv_v, [f1 * 3 + cc])
            v2 = plsc.load_gather(sv_v, [f2 * 3 + cc])
            v3 = plsc.load_gather(sv_v, [f3 * 3 + cc])
            bb_v[c, pl.ds(u * SC_LANES, SC_LANES)] = (v1 + v2 + v3) * third
            for j in range(NPF):
                w1c = w_v[0, j, pl.ds(o, SC_LANES)]
                w2c = w_v[1, j, pl.ds(o, SC_LANES)]
                w3c = w_v[2, j, pl.ds(o, SC_LANES)]
                pp_v[c, j, pl.ds(u * SC_LANES, SC_LANES)] = (
                    w1c * v1 + w2c * v2 + w3c * v3)
    for c in range(3):
        pltpu.sync_copy(bb_v.at[c], sb_h.at[pl.ds(c * F_SRC + base, f_per_w)])
        for j in range(NPF):
            pltpu.sync_copy(pp_v.at[c, j],
                            pt_h.at[pl.ds((c * NPF + j) * F_SRC + base, f_per_w)])

    tbase = wid * t_per_w
    for u in range(t_per_w // SC_LANES):
        o = tbase + u * SC_LANES
        t1 = tf_v[0, pl.ds(o, SC_LANES)]
        t2 = tf_v[1, pl.ds(o, SC_LANES)]
        t3 = tf_v[2, pl.ds(o, SC_LANES)]
        for c in range(3):
            cc = jnp.full((SC_LANES,), c, jnp.int32)
            g = (plsc.load_gather(tv_v, [t1 * 3 + cc])
                 + plsc.load_gather(tv_v, [t2 * 3 + cc])
                 + plsc.load_gather(tv_v, [t3 * 3 + cc]))
            tb_v[c, pl.ds(u * SC_LANES, SC_LANES)] = g * third
    for c in range(3):
        pltpu.sync_copy(tb_v.at[c], tb_h.at[pl.ds(c * N_TGT + tbase, t_per_w)])


def _sc_gather(svr, tvr, sft, tft, wst):
    n_sv = svr.shape[0]
    n_tv = tvr.shape[0]
    fn = pl.kernel(
        _sc_body,
        out_type=[
            jax.ShapeDtypeStruct((3 * F_SRC,), jnp.float32),
            jax.ShapeDtypeStruct((3 * NPF * F_SRC,), jnp.float32),
            jax.ShapeDtypeStruct((3 * N_TGT,), jnp.float32),
        ],
        mesh=plsc.VectorSubcoreMesh(core_axis_name="c", subcore_axis_name="s"),
        compiler_params=pltpu.CompilerParams(needs_layout_passes=False),
        scratch_types=[
            pltpu.VMEM((n_sv * 3,), jnp.float32),
            pltpu.VMEM((n_tv * 3,), jnp.float32),
            pltpu.VMEM((3, F_SRC), jnp.int32),
            pltpu.VMEM((3, N_TGT), jnp.int32),
            pltpu.VMEM((3, NPF, F_SRC), jnp.float32),
            pltpu.VMEM((3, F_SRC // SC_WORKERS), jnp.float32),
            pltpu.VMEM((3, NPF, F_SRC // SC_WORKERS), jnp.float32),
            pltpu.VMEM((3, N_TGT // SC_WORKERS), jnp.float32),
        ],
    )
    sb_f, pt_f, tb_f = fn(svr.reshape(-1), tvr.reshape(-1), sft, tft, wst)
    return (sb_f.reshape(3, F_SRC), pt_f.reshape(3, NPF, F_SRC),
            tb_f.reshape(3, N_TGT))


def _sample_weights():
    rk = jax.random.key(42)
    r1 = jnp.sqrt(jax.random.uniform(jax.random.fold_in(rk, 0), (F_SRC, NPF), dtype=jnp.float32))
    r2 = jax.random.uniform(jax.random.fold_in(rk, 1), (F_SRC, NPF), dtype=jnp.float32)
    w1 = 1.0 - r1
    w2 = r1 * (1.0 - r2)
    w3 = r1 * r2
    return w1, w2, w3


def kernel(source_vertices, source_faces, target_vertices, target_faces, face_probs):
    sv = source_vertices[0]
    tv = target_vertices[0]
    sf = source_faces.astype(jnp.int32)
    tf = target_faces.astype(jnp.int32)
    w1, w2, w3 = _sample_weights()

    wst = jnp.stack([w1.T, w2.T, w3.T])
    sb_t, pt_t, tb_t = _sc_gather(sv, tv, sf.T, tf, wst)
    points = jnp.transpose(pt_t, (2, 1, 0)).reshape(N_PTS, 3)

    px = jnp.zeros((N_PTS, 128), jnp.float32).at[:, 0:3].set(points)
    sb_col = jnp.zeros((F_SRC, 128), jnp.float32).at[:, 0:3].set(sb_t.T)
    sx_row = jnp.zeros((8, F_SRC), jnp.float32).at[0:3, :].set(sb_t)
    ty_row = jnp.zeros((8, N_TGT), jnp.float32).at[0:3, :].set(tb_t)
    sxbt = jnp.zeros((128, F_SRC), jnp.bfloat16).at[0:3, :].set(
        sb_t.astype(jnp.bfloat16))
    tybt = jnp.zeros((128, N_TGT), jnp.bfloat16).at[0:3, :].set(
        tb_t.astype(jnp.bfloat16))
    pxb = px.astype(jnp.bfloat16)
    sbb = sb_col.astype(jnp.bfloat16)
    pr_row = face_probs[None, :]
    pc_col = face_probs[:, None]
    py_col = jnp.repeat(face_probs, NPF)[:, None]

    loss = _dense_loss(px, pxb, sb_col, sbb, pc_col, py_col, sx_row, sxbt,
                       ty_row, tybt, pr_row)
    return loss[0, 0]


# speedup vs baseline: 17.0277x; 1.1026x over previous
import jax
import jax.numpy as jnp
from jax import lax
from jax.experimental import pallas as pl
from jax.experimental.pallas import tpu as pltpu
from jax.experimental.pallas import tpu_sc as plsc

SC_CORES = 2
SC_SUBCORES = 16
SC_WORKERS = SC_CORES * SC_SUBCORES
SC_LANES = 16

NPF = 4
KNN = 5
F_SRC = 2048
N_PTS = F_SRC * NPF
N_TGT = 4096
BLK_P = 512
BLK_F = BLK_P // NPF
GRID = N_PTS // BLK_P

IDX_BITS = 11
IDX_MASK = (1 << IDX_BITS) - 1


def _dense_body(px_ref, pxb_ref, sb_ref, sbb_ref, pc_ref, py_ref,
                sx_ref, sxbt_ref, ty_ref, tybt_ref, pr_ref, out_ref):
    b = pl.program_id(0)
    px = px_ref[...]
    sb = sb_ref[...]
    pc = pc_ref[...]
    py = py_ref[...]
    pr = pr_ref[...]

    px2 = jnp.zeros((BLK_P, 1), jnp.float32)
    sx2 = jnp.zeros((1, F_SRC), jnp.float32)
    sb2 = jnp.zeros((BLK_F, 1), jnp.float32)
    ty2 = jnp.zeros((1, N_TGT), jnp.float32)
    for c in range(3):
        px2 = px2 + px[:, c:c + 1] * px[:, c:c + 1]
        sxc = sx_ref[c:c + 1, :]
        sx2 = sx2 + sxc * sxc
        sb2 = sb2 + sb[:, c:c + 1] * sb[:, c:c + 1]
        tyc = ty_ref[c:c + 1, :]
        ty2 = ty2 + tyc * tyc

    cross = jnp.dot(pxb_ref[...], sxbt_ref[...],
                    preferred_element_type=jnp.float32)
    d = jnp.maximum(px2 + sx2 - 2.0 * cross, 0.0)

    ii = lax.broadcasted_iota(jnp.int32, (BLK_P, F_SRC), 1)
    key = ((d.view(jnp.int32) & ~IDX_MASK) | ii).view(jnp.float32)
    face_col = (lax.broadcasted_iota(jnp.int32, (BLK_P, 1), 0) + b * BLK_P) // NPF
    tot = jnp.zeros((BLK_P, 1), jnp.float32)
    s_self = jnp.zeros((BLK_P, 1), jnp.float32)
    has_self = jnp.zeros((BLK_P, 1), jnp.bool_)
    last_pd = jnp.zeros((BLK_P, 1), jnp.float32)
    inf = jnp.float32(jnp.inf)
    for i in range(KNN + 1):
        kmin = jnp.min(key, axis=1, keepdims=True)
        oh = key == kmin
        p = jnp.sum(jnp.where(oh, pr, 0.0), axis=1, keepdims=True)
        kbits = kmin.view(jnp.int32)
        dq = (kbits & ~IDX_MASK).view(jnp.float32)
        idx = kbits & IDX_MASK
        pd = p * dq
        tot = tot + pd
        selfhit = idx == face_col
        s_self = s_self + jnp.where(selfhit, pd, 0.0)
        has_self = has_self | selfhit
        if i == KNN:
            last_pd = pd
        else:
            key = jnp.where(oh, inf, key)
    mean_term = jnp.where(has_self, tot - s_self, tot - last_pd) * (1.0 / KNN)

    ct = jnp.dot(pxb_ref[...], tybt_ref[...],
                 preferred_element_type=jnp.float32)
    dt = jnp.maximum(px2 + ty2 - 2.0 * ct, 0.0)
    mt = jnp.min(dt, axis=1, keepdims=True)
    cf = jnp.dot(sbb_ref[...], tybt_ref[...],
                 preferred_element_type=jnp.float32)
    df = jnp.maximum(sb2 + ty2 - 2.0 * cf, 0.0)
    fmin = jnp.min(df, axis=1, keepdims=True)

    rev = jnp.sum(py * mt + (1.0 - py) * mean_term)
    fwd = jnp.sum(pc * fmin)

    @pl.when(b == 0)
    def _():
        out_ref[...] = jnp.zeros((1, 1), jnp.float32)

    out_ref[...] += (rev + fwd).reshape(1, 1)


def _dense_loss(px, pxb, sb_col, sbb, pc_col, py_col, sx_row, sxbt, ty_row,
                tybt, pr_row):
    return pl.pallas_call(
        _dense_body,
        grid=(GRID,),
        in_specs=[
            pl.BlockSpec((BLK_P, 128), lambda b: (b, 0)),
            pl.BlockSpec((BLK_P, 128), lambda b: (b, 0)),
            pl.BlockSpec((BLK_F, 128), lambda b: (b, 0)),
            pl.BlockSpec((BLK_F, 128), lambda b: (b, 0)),
            pl.BlockSpec((BLK_F, 1), lambda b: (b, 0)),
            pl.BlockSpec((BLK_P, 1), lambda b: (b, 0)),
            pl.BlockSpec((8, F_SRC), lambda b: (0, 0)),
            pl.BlockSpec((128, F_SRC), lambda b: (0, 0)),
            pl.BlockSpec((8, N_TGT), lambda b: (0, 0)),
            pl.BlockSpec((128, N_TGT), lambda b: (0, 0)),
            pl.BlockSpec((1, F_SRC), lambda b: (0, 0)),
        ],
        out_specs=pl.BlockSpec((1, 1), lambda b: (0, 0)),
        out_shape=jax.ShapeDtypeStruct((1, 1), jnp.float32),
    )(px, pxb, sb_col, sbb, pc_col, py_col, sx_row, sxbt, ty_row, tybt, pr_row)


def _sc_body(sv_h, tv_h, sf_h, tf_h, w_h, sb_h, pt_h, tb_h,
             sv_v, tv_v, sf_v, tf_v, w_v, bb_v, pp_v, tb_v):
    wid = lax.axis_index("s") * SC_CORES + lax.axis_index("c")
    pltpu.sync_copy(sv_h, sv_v)
    pltpu.sync_copy(tv_h, tv_v)
    pltpu.sync_copy(sf_h, sf_v)
    pltpu.sync_copy(tf_h, tf_v)
    pltpu.sync_copy(w_h, w_v)

    f_per_w = F_SRC // SC_WORKERS
    t_per_w = N_TGT // SC_WORKERS
    base = wid * f_per_w
    third = jnp.float32(1.0 / 3.0)
    for u in range(f_per_w // SC_LANES):
        o = base + u * SC_LANES
        f1 = sf_v[0, pl.ds(o, SC_LANES)]
        f2 = sf_v[1, pl.ds(o, SC_LANES)]
        f3 = sf_v[2, pl.ds(o, SC_LANES)]
        for c in range(3):
            cc = jnp.full((SC_LANES,), c, jnp.int32)
            v1 = plsc.load_gather(sv_v, [f1 * 3 + cc])
            v2 = plsc.load_gather(sv_v, [f2 * 3 + cc])
            v3 = plsc.load_gather(sv_v, [f3 * 3 + cc])
            bb_v[c, pl.ds(u * SC_LANES, SC_LANES)] = (v1 + v2 + v3) * third
            for j in range(NPF):
                w1c = w_v[0, j, pl.ds(o, SC_LANES)]
                w2c = w_v[1, j, pl.ds(o, SC_LANES)]
                w3c = w_v[2, j, pl.ds(o, SC_LANES)]
                pp_v[c, j, pl.ds(u * SC_LANES, SC_LANES)] = (
                    w1c * v1 + w2c * v2 + w3c * v3)
    for c in range(3):
        pltpu.sync_copy(bb_v.at[c], sb_h.at[pl.ds(c * F_SRC + base, f_per_w)])
        for j in range(NPF):
            pltpu.sync_copy(pp_v.at[c, j],
                            pt_h.at[pl.ds((c * NPF + j) * F_SRC + base, f_per_w)])

    tbase = wid * t_per_w
    for u in range(t_per_w // SC_LANES):
        o = tbase + u * SC_LANES
        t1 = tf_v[0, pl.ds(o, SC_LANES)]
        t2 = tf_v[1, pl.ds(o, SC_LANES)]
        t3 = tf_v[2, pl.ds(o, SC_LANES)]
        for c in range(3):
            cc = jnp.full((SC_LANES,), c, jnp.int32)
            g = (plsc.load_gather(tv_v, [t1 * 3 + cc])
                 + plsc.load_gather(tv_v, [t2 * 3 + cc])
                 + plsc.load_gather(tv_v, [t3 * 3 + cc]))
            tb_v[c, pl.ds(u * SC_LANES, SC_LANES)] = g * third
    for c in range(3):
        pltpu.sync_copy(tb_v.at[c], tb_h.at[pl.ds(c * N_TGT + tbase, t_per_w)])


def _sc_gather(svr, tvr, sft, tft, wst):
    n_sv = svr.shape[0]
    n_tv = tvr.shape[0]
    fn = pl.kernel(
        _sc_body,
        out_type=[
            jax.ShapeDtypeStruct((3 * F_SRC,), jnp.float32),
            jax.ShapeDtypeStruct((3 * NPF * F_SRC,), jnp.float32),
            jax.ShapeDtypeStruct((3 * N_TGT,), jnp.float32),
        ],
        mesh=plsc.VectorSubcoreMesh(core_axis_name="c", subcore_axis_name="s"),
        compiler_params=pltpu.CompilerParams(needs_layout_passes=False),
        scratch_types=[
            pltpu.VMEM((n_sv * 3,), jnp.float32),
            pltpu.VMEM((n_tv * 3,), jnp.float32),
            pltpu.VMEM((3, F_SRC), jnp.int32),
            pltpu.VMEM((3, N_TGT), jnp.int32),
            pltpu.VMEM((3, NPF, F_SRC), jnp.float32),
            pltpu.VMEM((3, F_SRC // SC_WORKERS), jnp.float32),
            pltpu.VMEM((3, NPF, F_SRC // SC_WORKERS), jnp.float32),
            pltpu.VMEM((3, N_TGT // SC_WORKERS), jnp.float32),
        ],
    )
    sb_f, pt_f, tb_f = fn(svr.reshape(-1), tvr.reshape(-1), sft, tft, wst)
    return (sb_f.reshape(3, F_SRC), pt_f.reshape(3, NPF, F_SRC),
            tb_f.reshape(3, N_TGT))


def _sample_weights():
    rk = jax.random.key(42)
    r1 = jnp.sqrt(jax.random.uniform(jax.random.fold_in(rk, 0), (F_SRC, NPF), dtype=jnp.float32))
    r2 = jax.random.uniform(jax.random.fold_in(rk, 1), (F_SRC, NPF), dtype=jnp.float32)
    w1 = 1.0 - r1
    w2 = r1 * (1.0 - r2)
    w3 = r1 * r2
    return w1, w2, w3


def kernel(source_vertices, source_faces, target_vertices, target_faces, face_probs):
    sv = source_vertices[0]
    tv = target_vertices[0]
    sf = source_faces.astype(jnp.int32)
    tf = target_faces.astype(jnp.int32)
    w1, w2, w3 = _sample_weights()

    wst = jnp.stack([w1.T, w2.T, w3.T])
    sb_t, pt_t, tb_t = _sc_gather(sv, tv, sf.T, tf, wst)
    points = jnp.transpose(pt_t, (2, 1, 0)).reshape(N_PTS, 3)

    px = jnp.zeros((N_PTS, 128), jnp.float32).at[:, 0:3].set(points)
    sb_col = jnp.zeros((F_SRC, 128), jnp.float32).at[:, 0:3].set(sb_t.T)
    sx_row = jnp.zeros((8, F_SRC), jnp.float32).at[0:3, :].set(sb_t)
    ty_row = jnp.zeros((8, N_TGT), jnp.float32).at[0:3, :].set(tb_t)
    sxbt = jnp.zeros((128, F_SRC), jnp.bfloat16).at[0:3, :].set(
        sb_t.astype(jnp.bfloat16))
    tybt = jnp.zeros((128, N_TGT), jnp.bfloat16).at[0:3, :].set(
        tb_t.astype(jnp.bfloat16))
    pxb = px.astype(jnp.bfloat16)
    sbb = sb_col.astype(jnp.bfloat16)
    pr_row = face_probs[None, :]
    pc_col = face_probs[:, None]
    py_col = jnp.repeat(face_probs, NPF)[:, None]

    loss = _dense_loss(px, pxb, sb_col, sbb, pc_col, py_col, sx_row, sxbt,
                       ty_row, tybt, pr_row)
    return loss[0, 0]


# speedup vs baseline: 17.7494x; 1.0424x over previous
import jax
import jax.numpy as jnp
from jax import lax
from jax.experimental import pallas as pl
from jax.experimental.pallas import tpu as pltpu
from jax.experimental.pallas import tpu_sc as plsc

SC_CORES = 2
SC_SUBCORES = 16
SC_WORKERS = SC_CORES * SC_SUBCORES
SC_LANES = 16

NPF = 4
KNN = 5
F_SRC = 2048
N_PTS = F_SRC * NPF
N_TGT = 4096
BLK_P = 512
BLK_F = BLK_P // NPF
GRID = N_PTS // BLK_P

IDX_BITS = 11
IDX_MASK = (1 << IDX_BITS) - 1


def _dense_body(px_ref, pxb_ref, sb_ref, sbb_ref, pc_ref, py_ref,
                sx_ref, sxbt_ref, ty_ref, tybt_ref, pr_ref, out_ref):
    b = pl.program_id(0)
    px = px_ref[...]
    sb = sb_ref[...]
    pc = pc_ref[...]
    py = py_ref[...]
    pr = pr_ref[...]

    px2 = jnp.zeros((BLK_P, 1), jnp.float32)
    sx2 = jnp.zeros((1, F_SRC), jnp.float32)
    sb2 = jnp.zeros((BLK_F, 1), jnp.float32)
    ty2 = jnp.zeros((1, N_TGT), jnp.float32)
    for c in range(3):
        px2 = px2 + px[:, c:c + 1] * px[:, c:c + 1]
        sxc = sx_ref[c:c + 1, :]
        sx2 = sx2 + sxc * sxc
        sb2 = sb2 + sb[:, c:c + 1] * sb[:, c:c + 1]
        tyc = ty_ref[c:c + 1, :]
        ty2 = ty2 + tyc * tyc

    cross = jnp.dot(pxb_ref[...], sxbt_ref[...],
                    preferred_element_type=jnp.float32)
    d = jnp.maximum(px2 + sx2 - 2.0 * cross, 0.0)

    ii = lax.broadcasted_iota(jnp.int32, (BLK_P, F_SRC), 1)
    key = ((d.view(jnp.int32) & ~IDX_MASK) | ii).view(jnp.float32)
    face_col = (lax.broadcasted_iota(jnp.int32, (BLK_P, 1), 0) + b * BLK_P) // NPF
    tot = jnp.zeros((BLK_P, 1), jnp.float32)
    s_self = jnp.zeros((BLK_P, 1), jnp.float32)
    has_self = jnp.zeros((BLK_P, 1), jnp.bool_)
    last_pd = jnp.zeros((BLK_P, 1), jnp.float32)
    inf = jnp.float32(jnp.inf)
    for i in range(KNN + 1):
        kmin = jnp.min(key, axis=1, keepdims=True)
        oh = key == kmin
        p = jnp.sum(jnp.where(oh, pr, 0.0), axis=1, keepdims=True)
        kbits = kmin.view(jnp.int32)
        dq = (kbits & ~IDX_MASK).view(jnp.float32)
        idx = kbits & IDX_MASK
        pd = p * dq
        tot = tot + pd
        selfhit = idx == face_col
        s_self = s_self + jnp.where(selfhit, pd, 0.0)
        has_self = has_self | selfhit
        if i == KNN:
            last_pd = pd
        else:
            key = jnp.where(oh, inf, key)
    mean_term = jnp.where(has_self, tot - s_self, tot - last_pd) * (1.0 / KNN)

    ct = jnp.dot(pxb_ref[...], tybt_ref[...],
                 preferred_element_type=jnp.float32)
    dt = jnp.maximum(px2 + ty2 - 2.0 * ct, 0.0)
    mt = jnp.min(dt, axis=1, keepdims=True)
    cf = jnp.dot(sbb_ref[...], tybt_ref[...],
                 preferred_element_type=jnp.float32)
    df = jnp.maximum(sb2 + ty2 - 2.0 * cf, 0.0)
    fmin = jnp.min(df, axis=1, keepdims=True)

    rev = jnp.sum(py * mt + (1.0 - py) * mean_term)
    fwd = jnp.sum(pc * fmin)

    @pl.when(b == 0)
    def _():
        out_ref[...] = jnp.zeros((1, 1), jnp.float32)

    out_ref[...] += (rev + fwd).reshape(1, 1)


def _dense_loss(px, pxb, sb_col, sbb, pc_col, py_col, sx_row, sxbt, ty_row,
                tybt, pr_row):
    return pl.pallas_call(
        _dense_body,
        grid=(GRID,),
        in_specs=[
            pl.BlockSpec((BLK_P, 128), lambda b: (b, 0)),
            pl.BlockSpec((BLK_P, 128), lambda b: (b, 0)),
            pl.BlockSpec((BLK_F, 128), lambda b: (b, 0)),
            pl.BlockSpec((BLK_F, 128), lambda b: (b, 0)),
            pl.BlockSpec((BLK_F, 1), lambda b: (b, 0)),
            pl.BlockSpec((BLK_P, 1), lambda b: (b, 0)),
            pl.BlockSpec((8, F_SRC), lambda b: (0, 0)),
            pl.BlockSpec((128, F_SRC), lambda b: (0, 0)),
            pl.BlockSpec((8, N_TGT), lambda b: (0, 0)),
            pl.BlockSpec((128, N_TGT), lambda b: (0, 0)),
            pl.BlockSpec((1, F_SRC), lambda b: (0, 0)),
        ],
        out_specs=pl.BlockSpec((1, 1), lambda b: (0, 0)),
        out_shape=jax.ShapeDtypeStruct((1, 1), jnp.float32),
    )(px, pxb, sb_col, sbb, pc_col, py_col, sx_row, sxbt, ty_row, tybt, pr_row)


def _sc_body(sv_h, tv_h, sf_h, tf_h, w_h, sb_h, pt_h, tb_h,
             sv_v, tv_v, sf_v, tf_v, w_v, bb_v, pp_v, tb_v):
    wid = lax.axis_index("s") * SC_CORES + lax.axis_index("c")
    pltpu.sync_copy(sv_h, sv_v)
    pltpu.sync_copy(tv_h, tv_v)
    pltpu.sync_copy(sf_h, sf_v)
    pltpu.sync_copy(tf_h, tf_v)
    pltpu.sync_copy(w_h, w_v)

    f_per_w = F_SRC // SC_WORKERS
    t_per_w = N_TGT // SC_WORKERS
    base = wid * f_per_w
    third = jnp.float32(1.0 / 3.0)
    for u in range(f_per_w // SC_LANES):
        o = base + u * SC_LANES
        f1 = sf_v[0, pl.ds(o, SC_LANES)]
        f2 = sf_v[1, pl.ds(o, SC_LANES)]
        f3 = sf_v[2, pl.ds(o, SC_LANES)]
        for c in range(3):
            cc = jnp.full((SC_LANES,), c, jnp.int32)
            v1 = plsc.load_gather(sv_v, [f1 * 3 + cc])
            v2 = plsc.load_gather(sv_v, [f2 * 3 + cc])
            v3 = plsc.load_gather(sv_v, [f3 * 3 + cc])
            bb_v[c, pl.ds(u * SC_LANES, SC_LANES)] = (v1 + v2 + v3) * third
            for j in range(NPF):
                w1c = w_v[0, j, pl.ds(o, SC_LANES)]
                w2c = w_v[1, j, pl.ds(o, SC_LANES)]
                w3c = w_v[2, j, pl.ds(o, SC_LANES)]
                pp_v[c, j, pl.ds(u * SC_LANES, SC_LANES)] = (
                    w1c * v1 + w2c * v2 + w3c * v3)
    for c in range(3):
        pltpu.sync_copy(bb_v.at[c], sb_h.at[pl.ds(c * F_SRC + base, f_per_w)])
        for j in range(NPF):
            pltpu.sync_copy(pp_v.at[c, j],
                            pt_h.at[pl.ds((c * NPF + j) * F_SRC + base, f_per_w)])

    tbase = wid * t_per_w
    for u in range(t_per_w // SC_LANES):
        o = tbase + u * SC_LANES
        t1 = tf_v[0, pl.ds(o, SC_LANES)]
        t2 = tf_v[1, pl.ds(o, SC_LANES)]
        t3 = tf_v[2, pl.ds(o, SC_LANES)]
        for c in range(3):
            cc = jnp.full((SC_LANES,), c, jnp.int32)
            g = (plsc.load_gather(tv_v, [t1 * 3 + cc])
                 + plsc.load_gather(tv_v, [t2 * 3 + cc])
                 + plsc.load_gather(tv_v, [t3 * 3 + cc]))
            tb_v[c, pl.ds(u * SC_LANES, SC_LANES)] = g * third
    for c in range(3):
        pltpu.sync_copy(tb_v.at[c], tb_h.at[pl.ds(c * N_TGT + tbase, t_per_w)])


def _sc_gather(svr, tvr, sft, tft, wst):
    n_sv = svr.shape[0]
    n_tv = tvr.shape[0]
    fn = pl.kernel(
        _sc_body,
        out_type=[
            jax.ShapeDtypeStruct((3 * F_SRC,), jnp.float32),
            jax.ShapeDtypeStruct((3 * NPF * F_SRC,), jnp.float32),
            jax.ShapeDtypeStruct((3 * N_TGT,), jnp.float32),
        ],
        mesh=plsc.VectorSubcoreMesh(core_axis_name="c", subcore_axis_name="s"),
        compiler_params=pltpu.CompilerParams(needs_layout_passes=False),
        scratch_types=[
            pltpu.VMEM((n_sv * 3,), jnp.float32),
            pltpu.VMEM((n_tv * 3,), jnp.float32),
            pltpu.VMEM((3, F_SRC), jnp.int32),
            pltpu.VMEM((3, N_TGT), jnp.int32),
            pltpu.VMEM((3, NPF, F_SRC), jnp.float32),
            pltpu.VMEM((3, F_SRC // SC_WORKERS), jnp.float32),
            pltpu.VMEM((3, NPF, F_SRC // SC_WORKERS), jnp.float32),
            pltpu.VMEM((3, N_TGT // SC_WORKERS), jnp.float32),
        ],
    )
    sb_f, pt_f, tb_f = fn(svr.reshape(-1), tvr.reshape(-1), sft, tft, wst)
    return (sb_f.reshape(3, F_SRC), pt_f.reshape(3, NPF, F_SRC),
            tb_f.reshape(3, N_TGT))


def _sample_weights():
    import numpy as np
    rk = jax.random.key(42)
    r1 = jnp.sqrt(jax.random.uniform(jax.random.fold_in(rk, 0), (F_SRC, NPF), dtype=jnp.float32))
    r2 = jax.random.uniform(jax.random.fold_in(rk, 1), (F_SRC, NPF), dtype=jnp.float32)
    w1 = 1.0 - r1
    w2 = r1 * (1.0 - r2)
    w3 = r1 * r2
    wst = jnp.stack([w1.T, w2.T, w3.T])
    return np.asarray(jax.device_get(wst))


_WST_CONST = _sample_weights()


def kernel(source_vertices, source_faces, target_vertices, target_faces, face_probs):
    sv = source_vertices[0]
    tv = target_vertices[0]
    sf = source_faces.astype(jnp.int32)
    tf = target_faces.astype(jnp.int32)
    sb_t, pt_t, tb_t = _sc_gather(sv, tv, sf.T, tf, _WST_CONST)
    points = jnp.transpose(pt_t, (2, 1, 0)).reshape(N_PTS, 3)

    px = jnp.zeros((N_PTS, 128), jnp.float32).at[:, 0:3].set(points)
    sb_col = jnp.zeros((F_SRC, 128), jnp.float32).at[:, 0:3].set(sb_t.T)
    sx_row = jnp.zeros((8, F_SRC), jnp.float32).at[0:3, :].set(sb_t)
    ty_row = jnp.zeros((8, N_TGT), jnp.float32).at[0:3, :].set(tb_t)
    sxbt = jnp.zeros((128, F_SRC), jnp.bfloat16).at[0:3, :].set(
        sb_t.astype(jnp.bfloat16))
    tybt = jnp.zeros((128, N_TGT), jnp.bfloat16).at[0:3, :].set(
        tb_t.astype(jnp.bfloat16))
    pxb = px.astype(jnp.bfloat16)
    sbb = sb_col.astype(jnp.bfloat16)
    pr_row = face_probs[None, :]
    pc_col = face_probs[:, None]
    py_col = jnp.repeat(face_probs, NPF)[:, None]

    loss = _dense_loss(px, pxb, sb_col, sbb, pc_col, py_col, sx_row, sxbt,
                       ty_row, tybt, pr_row)
    return loss[0, 0]


# speedup vs baseline: 17.9706x; 1.0125x over previous
import jax
import jax.numpy as jnp
from jax import lax
from jax.experimental import pallas as pl
from jax.experimental.pallas import tpu as pltpu
from jax.experimental.pallas import tpu_sc as plsc

SC_CORES = 2
SC_SUBCORES = 16
SC_WORKERS = SC_CORES * SC_SUBCORES
SC_LANES = 16

NPF = 4
KNN = 5
F_SRC = 2048
N_PTS = F_SRC * NPF
N_TGT = 4096
BLK_P = 1024
BLK_F = BLK_P // NPF
GRID = N_PTS // BLK_P

IDX_BITS = 11
IDX_MASK = (1 << IDX_BITS) - 1


def _dense_body(px_ref, pxb_ref, sb_ref, sbb_ref, pc_ref, py_ref,
                sx_ref, sxbt_ref, ty_ref, tybt_ref, pr_ref, out_ref):
    b = pl.program_id(0)
    px = px_ref[...]
    sb = sb_ref[...]
    pc = pc_ref[...]
    py = py_ref[...]
    pr = pr_ref[...]

    px2 = jnp.zeros((BLK_P, 1), jnp.float32)
    sx2 = jnp.zeros((1, F_SRC), jnp.float32)
    sb2 = jnp.zeros((BLK_F, 1), jnp.float32)
    ty2 = jnp.zeros((1, N_TGT), jnp.float32)
    for c in range(3):
        px2 = px2 + px[:, c:c + 1] * px[:, c:c + 1]
        sxc = sx_ref[c:c + 1, :]
        sx2 = sx2 + sxc * sxc
        sb2 = sb2 + sb[:, c:c + 1] * sb[:, c:c + 1]
        tyc = ty_ref[c:c + 1, :]
        ty2 = ty2 + tyc * tyc

    cross = jnp.dot(pxb_ref[...], sxbt_ref[...],
                    preferred_element_type=jnp.float32)
    d = jnp.maximum(px2 + sx2 - 2.0 * cross, 0.0)

    ii = lax.broadcasted_iota(jnp.int32, (BLK_P, F_SRC), 1)
    key = ((d.view(jnp.int32) & ~IDX_MASK) | ii).view(jnp.float32)
    face_col = (lax.broadcasted_iota(jnp.int32, (BLK_P, 1), 0) + b * BLK_P) // NPF
    tot = jnp.zeros((BLK_P, 1), jnp.float32)
    s_self = jnp.zeros((BLK_P, 1), jnp.float32)
    has_self = jnp.zeros((BLK_P, 1), jnp.bool_)
    last_pd = jnp.zeros((BLK_P, 1), jnp.float32)
    inf = jnp.float32(jnp.inf)
    for i in range(KNN + 1):
        kmin = jnp.min(key, axis=1, keepdims=True)
        oh = key == kmin
        p = jnp.sum(jnp.where(oh, pr, 0.0), axis=1, keepdims=True)
        kbits = kmin.view(jnp.int32)
        dq = (kbits & ~IDX_MASK).view(jnp.float32)
        idx = kbits & IDX_MASK
        pd = p * dq
        tot = tot + pd
        selfhit = idx == face_col
        s_self = s_self + jnp.where(selfhit, pd, 0.0)
        has_self = has_self | selfhit
        if i == KNN:
            last_pd = pd
        else:
            key = jnp.where(oh, inf, key)
    mean_term = jnp.where(has_self, tot - s_self, tot - last_pd) * (1.0 / KNN)

    ct = jnp.dot(pxb_ref[...], tybt_ref[...],
                 preferred_element_type=jnp.float32)
    dt = jnp.maximum(px2 + ty2 - 2.0 * ct, 0.0)
    mt = jnp.min(dt, axis=1, keepdims=True)
    cf = jnp.dot(sbb_ref[...], tybt_ref[...],
                 preferred_element_type=jnp.float32)
    df = jnp.maximum(sb2 + ty2 - 2.0 * cf, 0.0)
    fmin = jnp.min(df, axis=1, keepdims=True)

    rev = jnp.sum(py * mt + (1.0 - py) * mean_term)
    fwd = jnp.sum(pc * fmin)

    @pl.when(b == 0)
    def _():
        out_ref[...] = jnp.zeros((1, 1), jnp.float32)

    out_ref[...] += (rev + fwd).reshape(1, 1)


def _dense_loss(px, pxb, sb_col, sbb, pc_col, py_col, sx_row, sxbt, ty_row,
                tybt, pr_row):
    return pl.pallas_call(
        _dense_body,
        grid=(GRID,),
        in_specs=[
            pl.BlockSpec((BLK_P, 128), lambda b: (b, 0)),
            pl.BlockSpec((BLK_P, 128), lambda b: (b, 0)),
            pl.BlockSpec((BLK_F, 128), lambda b: (b, 0)),
            pl.BlockSpec((BLK_F, 128), lambda b: (b, 0)),
            pl.BlockSpec((BLK_F, 1), lambda b: (b, 0)),
            pl.BlockSpec((BLK_P, 1), lambda b: (b, 0)),
            pl.BlockSpec((8, F_SRC), lambda b: (0, 0)),
            pl.BlockSpec((128, F_SRC), lambda b: (0, 0)),
            pl.BlockSpec((8, N_TGT), lambda b: (0, 0)),
            pl.BlockSpec((128, N_TGT), lambda b: (0, 0)),
            pl.BlockSpec((1, F_SRC), lambda b: (0, 0)),
        ],
        out_specs=pl.BlockSpec((1, 1), lambda b: (0, 0)),
        out_shape=jax.ShapeDtypeStruct((1, 1), jnp.float32),
    )(px, pxb, sb_col, sbb, pc_col, py_col, sx_row, sxbt, ty_row, tybt, pr_row)


def _sc_body(sv_h, tv_h, sf_h, tf_h, w_h, sb_h, pt_h, tb_h,
             sv_v, tv_v, sf_v, tf_v, w_v, bb_v, pp_v, tb_v):
    wid = lax.axis_index("s") * SC_CORES + lax.axis_index("c")
    pltpu.sync_copy(sv_h, sv_v)
    pltpu.sync_copy(tv_h, tv_v)
    pltpu.sync_copy(sf_h, sf_v)
    pltpu.sync_copy(tf_h, tf_v)
    pltpu.sync_copy(w_h, w_v)

    f_per_w = F_SRC // SC_WORKERS
    t_per_w = N_TGT // SC_WORKERS
    base = wid * f_per_w
    third = jnp.float32(1.0 / 3.0)
    for u in range(f_per_w // SC_LANES):
        o = base + u * SC_LANES
        f1 = sf_v[0, pl.ds(o, SC_LANES)]
        f2 = sf_v[1, pl.ds(o, SC_LANES)]
        f3 = sf_v[2, pl.ds(o, SC_LANES)]
        for c in range(3):
            cc = jnp.full((SC_LANES,), c, jnp.int32)
            v1 = plsc.load_gather(sv_v, [f1 * 3 + cc])
            v2 = plsc.load_gather(sv_v, [f2 * 3 + cc])
            v3 = plsc.load_gather(sv_v, [f3 * 3 + cc])
            bb_v[c, pl.ds(u * SC_LANES, SC_LANES)] = (v1 + v2 + v3) * third
            for j in range(NPF):
                w1c = w_v[0, j, pl.ds(o, SC_LANES)]
                w2c = w_v[1, j, pl.ds(o, SC_LANES)]
                w3c = w_v[2, j, pl.ds(o, SC_LANES)]
                pp_v[c, j, pl.ds(u * SC_LANES, SC_LANES)] = (
                    w1c * v1 + w2c * v2 + w3c * v3)
    for c in range(3):
        pltpu.sync_copy(bb_v.at[c], sb_h.at[pl.ds(c * F_SRC + base, f_per_w)])
        for j in range(NPF):
            pltpu.sync_copy(pp_v.at[c, j],
                            pt_h.at[pl.ds((c * NPF + j) * F_SRC + base, f_per_w)])

    tbase = wid * t_per_w
    for u in range(t_per_w // SC_LANES):
        o = tbase + u * SC_LANES
        t1 = tf_v[0, pl.ds(o, SC_LANES)]
        t2 = tf_v[1, pl.ds(o, SC_LANES)]
        t3 = tf_v[2, pl.ds(o, SC_LANES)]
        for c in range(3):
            cc = jnp.full((SC_LANES,), c, jnp.int32)
            g = (plsc.load_gather(tv_v, [t1 * 3 + cc])
                 + plsc.load_gather(tv_v, [t2 * 3 + cc])
                 + plsc.load_gather(tv_v, [t3 * 3 + cc]))
            tb_v[c, pl.ds(u * SC_LANES, SC_LANES)] = g * third
    for c in range(3):
        pltpu.sync_copy(tb_v.at[c], tb_h.at[pl.ds(c * N_TGT + tbase, t_per_w)])


def _sc_gather(svr, tvr, sft, tft, wst):
    n_sv = svr.shape[0]
    n_tv = tvr.shape[0]
    fn = pl.kernel(
        _sc_body,
        out_type=[
            jax.ShapeDtypeStruct((3 * F_SRC,), jnp.float32),
            jax.ShapeDtypeStruct((3 * NPF * F_SRC,), jnp.float32),
            jax.ShapeDtypeStruct((3 * N_TGT,), jnp.float32),
        ],
        mesh=plsc.VectorSubcoreMesh(core_axis_name="c", subcore_axis_name="s"),
        compiler_params=pltpu.CompilerParams(needs_layout_passes=False),
        scratch_types=[
            pltpu.VMEM((n_sv * 3,), jnp.float32),
            pltpu.VMEM((n_tv * 3,), jnp.float32),
            pltpu.VMEM((3, F_SRC), jnp.int32),
            pltpu.VMEM((3, N_TGT), jnp.int32),
            pltpu.VMEM((3, NPF, F_SRC), jnp.float32),
            pltpu.VMEM((3, F_SRC // SC_WORKERS), jnp.float32),
            pltpu.VMEM((3, NPF, F_SRC // SC_WORKERS), jnp.float32),
            pltpu.VMEM((3, N_TGT // SC_WORKERS), jnp.float32),
        ],
    )
    sb_f, pt_f, tb_f = fn(svr.reshape(-1), tvr.reshape(-1), sft, tft, wst)
    return (sb_f.reshape(3, F_SRC), pt_f.reshape(3, NPF, F_SRC),
            tb_f.reshape(3, N_TGT))


def _sample_weights():
    import numpy as np
    rk = jax.random.key(42)
    r1 = jnp.sqrt(jax.random.uniform(jax.random.fold_in(rk, 0), (F_SRC, NPF), dtype=jnp.float32))
    r2 = jax.random.uniform(jax.random.fold_in(rk, 1), (F_SRC, NPF), dtype=jnp.float32)
    w1 = 1.0 - r1
    w2 = r1 * (1.0 - r2)
    w3 = r1 * r2
    wst = jnp.stack([w1.T, w2.T, w3.T])
    return np.asarray(jax.device_get(wst))


_WST_CONST = _sample_weights()


def kernel(source_vertices, source_faces, target_vertices, target_faces, face_probs):
    sv = source_vertices[0]
    tv = target_vertices[0]
    sf = source_faces.astype(jnp.int32)
    tf = target_faces.astype(jnp.int32)
    sb_t, pt_t, tb_t = _sc_gather(sv, tv, sf.T, tf, _WST_CONST)
    points = jnp.transpose(pt_t, (2, 1, 0)).reshape(N_PTS, 3)

    px = jnp.zeros((N_PTS, 128), jnp.float32).at[:, 0:3].set(points)
    sb_col = jnp.zeros((F_SRC, 128), jnp.float32).at[:, 0:3].set(sb_t.T)
    sx_row = jnp.zeros((8, F_SRC), jnp.float32).at[0:3, :].set(sb_t)
    ty_row = jnp.zeros((8, N_TGT), jnp.float32).at[0:3, :].set(tb_t)
    sxbt = jnp.zeros((128, F_SRC), jnp.bfloat16).at[0:3, :].set(
        sb_t.astype(jnp.bfloat16))
    tybt = jnp.zeros((128, N_TGT), jnp.bfloat16).at[0:3, :].set(
        tb_t.astype(jnp.bfloat16))
    pxb = px.astype(jnp.bfloat16)
    sbb = sb_col.astype(jnp.bfloat16)
    pr_row = face_probs[None, :]
    pc_col = face_probs[:, None]
    py_col = jnp.repeat(face_probs, NPF)[:, None]

    loss = _dense_loss(px, pxb, sb_col, sbb, pc_col, py_col, sx_row, sxbt,
                       ty_row, tybt, pr_row)
    return loss[0, 0]


# speedup vs baseline: 18.4829x; 1.0285x over previous
import jax
import jax.numpy as jnp
from jax import lax
from jax.experimental import pallas as pl
from jax.experimental.pallas import tpu as pltpu
from jax.experimental.pallas import tpu_sc as plsc

SC_CORES = 2
SC_SUBCORES = 16
SC_WORKERS = SC_CORES * SC_SUBCORES
SC_LANES = 16

NPF = 4
KNN = 5
F_SRC = 2048
N_PTS = F_SRC * NPF
N_TGT = 4096
BLK_P = 1024
BLK_F = BLK_P // NPF
GRID = N_PTS // BLK_P

IDX_BITS = 11
IDX_MASK = (1 << IDX_BITS) - 1


def _dense_body(pxb_ref, sbb_ref, pc_ref, py_ref, px2_ref, sx2_ref, sb2_ref,
                ty2_ref, sxbt_ref, tybt_ref, pr_ref, out_ref):
    b = pl.program_id(0)
    pc = pc_ref[...]
    py = py_ref[...]
    pr = pr_ref[...]
    px2 = px2_ref[...]
    sx2 = sx2_ref[...]
    sb2 = sb2_ref[...]
    ty2 = ty2_ref[...]

    cross = jnp.dot(pxb_ref[...], sxbt_ref[...],
                    preferred_element_type=jnp.float32)
    d = jnp.maximum(px2 + sx2 - 2.0 * cross, 0.0)

    ii = lax.broadcasted_iota(jnp.int32, (BLK_P, F_SRC), 1)
    key = ((d.view(jnp.int32) & ~IDX_MASK) | ii).view(jnp.float32)
    face_col = (lax.broadcasted_iota(jnp.int32, (BLK_P, 1), 0) + b * BLK_P) // NPF
    tot = jnp.zeros((BLK_P, 1), jnp.float32)
    s_self = jnp.zeros((BLK_P, 1), jnp.float32)
    has_self = jnp.zeros((BLK_P, 1), jnp.bool_)
    last_pd = jnp.zeros((BLK_P, 1), jnp.float32)
    inf = jnp.float32(jnp.inf)
    for i in range(KNN + 1):
        kmin = jnp.min(key, axis=1, keepdims=True)
        oh = key == kmin
        p = jnp.sum(jnp.where(oh, pr, 0.0), axis=1, keepdims=True)
        kbits = kmin.view(jnp.int32)
        dq = (kbits & ~IDX_MASK).view(jnp.float32)
        idx = kbits & IDX_MASK
        pd = p * dq
        tot = tot + pd
        selfhit = idx == face_col
        s_self = s_self + jnp.where(selfhit, pd, 0.0)
        has_self = has_self | selfhit
        if i == KNN:
            last_pd = pd
        else:
            key = jnp.where(oh, inf, key)
    mean_term = jnp.where(has_self, tot - s_self, tot - last_pd) * (1.0 / KNN)

    ct = jnp.dot(pxb_ref[...], tybt_ref[...],
                 preferred_element_type=jnp.float32)
    dt = jnp.maximum(px2 + ty2 - 2.0 * ct, 0.0)
    mt = jnp.min(dt, axis=1, keepdims=True)
    cf = jnp.dot(sbb_ref[...], tybt_ref[...],
                 preferred_element_type=jnp.float32)
    df = jnp.maximum(sb2 + ty2 - 2.0 * cf, 0.0)
    fmin = jnp.min(df, axis=1, keepdims=True)

    rev = jnp.sum(py * mt + (1.0 - py) * mean_term)
    fwd = jnp.sum(pc * fmin)

    @pl.when(b == 0)
    def _():
        out_ref[...] = jnp.zeros((1, 1), jnp.float32)

    out_ref[...] += (rev + fwd).reshape(1, 1)


def _dense_loss(pxb, sbb, pc_col, py_col, px2c, sx2r, sb2c, ty2r, sxbt,
                tybt, pr_row):
    return pl.pallas_call(
        _dense_body,
        grid=(GRID,),
        in_specs=[
            pl.BlockSpec((BLK_P, 128), lambda b: (b, 0)),
            pl.BlockSpec((BLK_F, 128), lambda b: (b, 0)),
            pl.BlockSpec((BLK_F, 1), lambda b: (b, 0)),
            pl.BlockSpec((BLK_P, 1), lambda b: (b, 0)),
            pl.BlockSpec((BLK_P, 1), lambda b: (b, 0)),
            pl.BlockSpec((1, F_SRC), lambda b: (0, 0)),
            pl.BlockSpec((BLK_F, 1), lambda b: (b, 0)),
            pl.BlockSpec((1, N_TGT), lambda b: (0, 0)),
            pl.BlockSpec((128, F_SRC), lambda b: (0, 0)),
            pl.BlockSpec((128, N_TGT), lambda b: (0, 0)),
            pl.BlockSpec((1, F_SRC), lambda b: (0, 0)),
        ],
        out_specs=pl.BlockSpec((1, 1), lambda b: (0, 0)),
        out_shape=jax.ShapeDtypeStruct((1, 1), jnp.float32),
    )(pxb, sbb, pc_col, py_col, px2c, sx2r, sb2c, ty2r, sxbt, tybt, pr_row)


def _sc_body(sv_h, tv_h, sf_h, tf_h, w_h, sb_h, pt_h, tb_h, sq_h, pq_h, tq_h,
             sv_v, tv_v, sf_v, tf_v, w_v, bb_v, pp_v, tb_v, bq_v, pq_v, tq_v):
    wid = lax.axis_index("s") * SC_CORES + lax.axis_index("c")
    pltpu.sync_copy(sv_h, sv_v)
    pltpu.sync_copy(tv_h, tv_v)
    pltpu.sync_copy(sf_h, sf_v)
    pltpu.sync_copy(tf_h, tf_v)
    pltpu.sync_copy(w_h, w_v)

    f_per_w = F_SRC // SC_WORKERS
    t_per_w = N_TGT // SC_WORKERS
    base = wid * f_per_w
    third = jnp.float32(1.0 / 3.0)
    lane4 = jnp.arange(SC_LANES, dtype=jnp.int32) * NPF
    for u in range(f_per_w // SC_LANES):
        o = base + u * SC_LANES
        f1 = sf_v[0, pl.ds(o, SC_LANES)]
        f2 = sf_v[1, pl.ds(o, SC_LANES)]
        f3 = sf_v[2, pl.ds(o, SC_LANES)]
        bsq = jnp.zeros((SC_LANES,), jnp.float32)
        psq = [jnp.zeros((SC_LANES,), jnp.float32) for _ in range(NPF)]
        for c in range(3):
            cc = jnp.full((SC_LANES,), c, jnp.int32)
            v1 = plsc.load_gather(sv_v, [f1 * 3 + cc])
            v2 = plsc.load_gather(sv_v, [f2 * 3 + cc])
            v3 = plsc.load_gather(sv_v, [f3 * 3 + cc])
            bary = (v1 + v2 + v3) * third
            bb_v[c, pl.ds(u * SC_LANES, SC_LANES)] = bary
            bsq = bsq + bary * bary
            for j in range(NPF):
                w1c = w_v[0, j, pl.ds(o, SC_LANES)]
                w2c = w_v[1, j, pl.ds(o, SC_LANES)]
                w3c = w_v[2, j, pl.ds(o, SC_LANES)]
                pt = w1c * v1 + w2c * v2 + w3c * v3
                pp_v[c, j, pl.ds(u * SC_LANES, SC_LANES)] = pt
                psq[j] = psq[j] + pt * pt
        bq_v[pl.ds(u * SC_LANES, SC_LANES)] = bsq
        for j in range(NPF):
            plsc.store_scatter(
                pq_v, [lane4 + (u * SC_LANES * NPF + j)], psq[j])
    for c in range(3):
        pltpu.sync_copy(bb_v.at[c], sb_h.at[pl.ds(c * F_SRC + base, f_per_w)])
        for j in range(NPF):
            pltpu.sync_copy(pp_v.at[c, j],
                            pt_h.at[pl.ds((c * NPF + j) * F_SRC + base, f_per_w)])
    pltpu.sync_copy(bq_v, sq_h.at[pl.ds(base, f_per_w)])
    pltpu.sync_copy(pq_v, pq_h.at[pl.ds(base * NPF, f_per_w * NPF)])

    tbase = wid * t_per_w
    for u in range(t_per_w // SC_LANES):
        o = tbase + u * SC_LANES
        t1 = tf_v[0, pl.ds(o, SC_LANES)]
        t2 = tf_v[1, pl.ds(o, SC_LANES)]
        t3 = tf_v[2, pl.ds(o, SC_LANES)]
        tsq = jnp.zeros((SC_LANES,), jnp.float32)
        for c in range(3):
            cc = jnp.full((SC_LANES,), c, jnp.int32)
            g = (plsc.load_gather(tv_v, [t1 * 3 + cc])
                 + plsc.load_gather(tv_v, [t2 * 3 + cc])
                 + plsc.load_gather(tv_v, [t3 * 3 + cc])) * third
            tb_v[c, pl.ds(u * SC_LANES, SC_LANES)] = g
            tsq = tsq + g * g
        tq_v[pl.ds(u * SC_LANES, SC_LANES)] = tsq
    for c in range(3):
        pltpu.sync_copy(tb_v.at[c], tb_h.at[pl.ds(c * N_TGT + tbase, t_per_w)])
    pltpu.sync_copy(tq_v, tq_h.at[pl.ds(tbase, t_per_w)])


def _sc_gather(svr, tvr, sft, tft, wst):
    n_sv = svr.shape[0]
    n_tv = tvr.shape[0]
    fn = pl.kernel(
        _sc_body,
        out_type=[
            jax.ShapeDtypeStruct((3 * F_SRC,), jnp.float32),
            jax.ShapeDtypeStruct((3 * NPF * F_SRC,), jnp.float32),
            jax.ShapeDtypeStruct((3 * N_TGT,), jnp.float32),
            jax.ShapeDtypeStruct((F_SRC,), jnp.float32),
            jax.ShapeDtypeStruct((N_PTS,), jnp.float32),
            jax.ShapeDtypeStruct((N_TGT,), jnp.float32),
        ],
        mesh=plsc.VectorSubcoreMesh(core_axis_name="c", subcore_axis_name="s"),
        compiler_params=pltpu.CompilerParams(needs_layout_passes=False),
        scratch_types=[
            pltpu.VMEM((n_sv * 3,), jnp.float32),
            pltpu.VMEM((n_tv * 3,), jnp.float32),
            pltpu.VMEM((3, F_SRC), jnp.int32),
            pltpu.VMEM((3, N_TGT), jnp.int32),
            pltpu.VMEM((3, NPF, F_SRC), jnp.float32),
            pltpu.VMEM((3, F_SRC // SC_WORKERS), jnp.float32),
            pltpu.VMEM((3, NPF, F_SRC // SC_WORKERS), jnp.float32),
            pltpu.VMEM((3, N_TGT // SC_WORKERS), jnp.float32),
            pltpu.VMEM((F_SRC // SC_WORKERS,), jnp.float32),
            pltpu.VMEM((NPF * F_SRC // SC_WORKERS,), jnp.float32),
            pltpu.VMEM((N_TGT // SC_WORKERS,), jnp.float32),
        ],
    )
    sb_f, pt_f, tb_f, sq_f, pq_f, tq_f = fn(
        svr.reshape(-1), tvr.reshape(-1), sft, tft, wst)
    return (sb_f.reshape(3, F_SRC), pt_f.reshape(3, NPF, F_SRC),
            tb_f.reshape(3, N_TGT), sq_f, pq_f, tq_f)


def _sample_weights():
    import numpy as np
    rk = jax.random.key(42)
    r1 = jnp.sqrt(jax.random.uniform(jax.random.fold_in(rk, 0), (F_SRC, NPF), dtype=jnp.float32))
    r2 = jax.random.uniform(jax.random.fold_in(rk, 1), (F_SRC, NPF), dtype=jnp.float32)
    w1 = 1.0 - r1
    w2 = r1 * (1.0 - r2)
    w3 = r1 * r2
    wst = jnp.stack([w1.T, w2.T, w3.T])
    return np.asarray(jax.device_get(wst))


_WST_CONST = _sample_weights()


def kernel(source_vertices, source_faces, target_vertices, target_faces, face_probs):
    sv = source_vertices[0]
    tv = target_vertices[0]
    sf = source_faces.astype(jnp.int32)
    tf = target_faces.astype(jnp.int32)
    sb_t, pt_t, tb_t, sq_f, pq_f, tq_f = _sc_gather(sv, tv, sf.T, tf,
                                                    _WST_CONST)
    points = jnp.transpose(pt_t, (2, 1, 0)).reshape(N_PTS, 3)

    pxb = jnp.zeros((N_PTS, 128), jnp.bfloat16).at[:, 0:3].set(
        points.astype(jnp.bfloat16))
    sbb = jnp.zeros((F_SRC, 128), jnp.bfloat16).at[:, 0:3].set(
        sb_t.T.astype(jnp.bfloat16))
    sxbt = jnp.zeros((128, F_SRC), jnp.bfloat16).at[0:3, :].set(
        sb_t.astype(jnp.bfloat16))
    tybt = jnp.zeros((128, N_TGT), jnp.bfloat16).at[0:3, :].set(
        tb_t.astype(jnp.bfloat16))
    px2c = pq_f[:, None]
    sx2r = sq_f[None, :]
    sb2c = sq_f[:, None]
    ty2r = tq_f[None, :]
    pr_row = face_probs[None, :]
    pc_col = face_probs[:, None]
    py_col = jnp.repeat(face_probs, NPF)[:, None]

    loss = _dense_loss(pxb, sbb, pc_col, py_col, px2c, sx2r, sb2c, ty2r,
                       sxbt, tybt, pr_row)
    return loss[0, 0]


# speedup vs baseline: 18.5061x; 1.0013x over previous
import jax
import jax.numpy as jnp
from jax import lax
from jax.experimental import pallas as pl
from jax.experimental.pallas import tpu as pltpu
from jax.experimental.pallas import tpu_sc as plsc

SC_CORES = 2
SC_SUBCORES = 16
SC_WORKERS = SC_CORES * SC_SUBCORES
SC_LANES = 16

NPF = 4
KNN = 5
F_SRC = 2048
N_PTS = F_SRC * NPF
N_TGT = 4096
BLK_P = 1024
BLK_F = BLK_P // NPF
GRID = N_PTS // BLK_P

IDX_BITS = 11
IDX_MASK = (1 << IDX_BITS) - 1


def _dense_body(pxb_ref, sbb_ref, pc_ref, py_ref, px2_ref, sx2_ref, sb2_ref,
                ty2_ref, sxbt_ref, tybt_ref, pr_ref, out_ref):
    b = pl.program_id(0)
    pc = pc_ref[...]
    py = py_ref[...]
    pr = pr_ref[...]
    px2 = px2_ref[...]
    sx2 = sx2_ref[...]
    sb2 = sb2_ref[...]
    ty2 = ty2_ref[...]

    cross = jnp.dot(pxb_ref[...], sxbt_ref[...],
                    preferred_element_type=jnp.float32)
    d = jnp.maximum(px2 + sx2 - 2.0 * cross, 0.0)

    ii = lax.broadcasted_iota(jnp.int32, (BLK_P, F_SRC), 1)
    key = ((d.view(jnp.int32) & ~IDX_MASK) | ii).view(jnp.float32)
    face_col = (lax.broadcasted_iota(jnp.int32, (BLK_P, 1), 0) + b * BLK_P) // NPF
    tot = jnp.zeros((BLK_P, 1), jnp.float32)
    s_self = jnp.zeros((BLK_P, 1), jnp.float32)
    has_self = jnp.zeros((BLK_P, 1), jnp.bool_)
    last_pd = jnp.zeros((BLK_P, 1), jnp.float32)
    inf = jnp.float32(jnp.inf)
    for i in range(KNN + 1):
        kmin = jnp.min(key, axis=1, keepdims=True)
        oh = key == kmin
        p = jnp.sum(jnp.where(oh, pr, 0.0), axis=1, keepdims=True)
        kbits = kmin.view(jnp.int32)
        dq = (kbits & ~IDX_MASK).view(jnp.float32)
        idx = kbits & IDX_MASK
        pd = p * dq
        tot = tot + pd
        selfhit = idx == face_col
        s_self = s_self + jnp.where(selfhit, pd, 0.0)
        has_self = has_self | selfhit
        if i == KNN:
            last_pd = pd
        else:
            key = jnp.where(oh, inf, key)
    mean_term = jnp.where(has_self, tot - s_self, tot - last_pd) * (1.0 / KNN)

    ct = jnp.dot(pxb_ref[...], tybt_ref[...],
                 preferred_element_type=jnp.float32)
    dt = jnp.maximum(px2 + ty2 - 2.0 * ct, 0.0)
    mt = jnp.min(dt, axis=1, keepdims=True)
    cf = jnp.dot(sbb_ref[...], tybt_ref[...],
                 preferred_element_type=jnp.float32)
    df = jnp.maximum(sb2 + ty2 - 2.0 * cf, 0.0)
    fmin = jnp.min(df, axis=1, keepdims=True)

    rev = jnp.sum(py * mt + (1.0 - py) * mean_term)
    fwd = jnp.sum(pc * fmin)

    @pl.when(b == 0)
    def _():
        out_ref[...] = jnp.zeros((1, 1), jnp.float32)

    out_ref[...] += (rev + fwd).reshape(1, 1)


def _dense_loss(pxb, sbb, pc_col, py_col, px2c, sx2r, sb2c, ty2r, sxbt,
                tybt, pr_row):
    return pl.pallas_call(
        _dense_body,
        grid=(GRID,),
        in_specs=[
            pl.BlockSpec((BLK_P, 128), lambda b: (b, 0)),
            pl.BlockSpec((BLK_F, 128), lambda b: (b, 0)),
            pl.BlockSpec((BLK_F, 1), lambda b: (b, 0)),
            pl.BlockSpec((BLK_P, 1), lambda b: (b, 0)),
            pl.BlockSpec((BLK_P, 1), lambda b: (b, 0)),
            pl.BlockSpec((1, F_SRC), lambda b: (0, 0)),
            pl.BlockSpec((BLK_F, 1), lambda b: (b, 0)),
            pl.BlockSpec((1, N_TGT), lambda b: (0, 0)),
            pl.BlockSpec((128, F_SRC), lambda b: (0, 0)),
            pl.BlockSpec((128, N_TGT), lambda b: (0, 0)),
            pl.BlockSpec((1, F_SRC), lambda b: (0, 0)),
        ],
        out_specs=pl.BlockSpec((1, 1), lambda b: (0, 0)),
        out_shape=jax.ShapeDtypeStruct((1, 1), jnp.float32),
    )(pxb, sbb, pc_col, py_col, px2c, sx2r, sb2c, ty2r, sxbt, tybt, pr_row)


def _sc_body(sv_h, tv_h, sf_h, tf_h, w_h, sb_h, pt_h, tb_h, sq_h, pq_h, tq_h,
             sv_v, tv_v, sf_v, tf_v, w_v, bb_v, pp_v, tb_v, bq_v, pq_v, tq_v):
    wid = lax.axis_index("s") * SC_CORES + lax.axis_index("c")
    pltpu.sync_copy(sv_h, sv_v)
    pltpu.sync_copy(tv_h, tv_v)
    pltpu.sync_copy(sf_h, sf_v)
    pltpu.sync_copy(tf_h, tf_v)
    pltpu.sync_copy(w_h, w_v)

    f_per_w = F_SRC // SC_WORKERS
    t_per_w = N_TGT // SC_WORKERS
    base = wid * f_per_w
    third = jnp.float32(1.0 / 3.0)
    lane4 = jnp.arange(SC_LANES, dtype=jnp.int32) * NPF
    for u in range(f_per_w // SC_LANES):
        o = base + u * SC_LANES
        f1 = sf_v[0, pl.ds(o, SC_LANES)]
        f2 = sf_v[1, pl.ds(o, SC_LANES)]
        f3 = sf_v[2, pl.ds(o, SC_LANES)]
        bsq = jnp.zeros((SC_LANES,), jnp.float32)
        psq = [jnp.zeros((SC_LANES,), jnp.float32) for _ in range(NPF)]
        for c in range(3):
            cc = jnp.full((SC_LANES,), c, jnp.int32)
            v1 = plsc.load_gather(sv_v, [f1 * 3 + cc])
            v2 = plsc.load_gather(sv_v, [f2 * 3 + cc])
            v3 = plsc.load_gather(sv_v, [f3 * 3 + cc])
            bary = (v1 + v2 + v3) * third
            bb_v[c, pl.ds(u * SC_LANES, SC_LANES)] = bary
            bsq = bsq + bary * bary
            for j in range(NPF):
                w1c = w_v[0, j, pl.ds(o, SC_LANES)]
                w2c = w_v[1, j, pl.ds(o, SC_LANES)]
                w3c = w_v[2, j, pl.ds(o, SC_LANES)]
                pt = w1c * v1 + w2c * v2 + w3c * v3
                pp_v[c, j, pl.ds(u * SC_LANES, SC_LANES)] = pt
                psq[j] = psq[j] + pt * pt
        bq_v[pl.ds(u * SC_LANES, SC_LANES)] = bsq
        for j in range(NPF):
            plsc.store_scatter(
                pq_v, [lane4 + (u * SC_LANES * NPF + j)], psq[j])
    for c in range(3):
        pltpu.sync_copy(bb_v.at[c], sb_h.at[pl.ds(c * F_SRC + base, f_per_w)])
        for j in range(NPF):
            pltpu.sync_copy(pp_v.at[c, j],
                            pt_h.at[pl.ds((c * NPF + j) * F_SRC + base, f_per_w)])
    pltpu.sync_copy(bq_v, sq_h.at[pl.ds(base, f_per_w)])
    pltpu.sync_copy(pq_v, pq_h.at[pl.ds(base * NPF, f_per_w * NPF)])

    tbase = wid * t_per_w
    for u in range(t_per_w // SC_LANES):
        o = tbase + u * SC_LANES
        t1 = tf_v[0, pl.ds(o, SC_LANES)]
        t2 = tf_v[1, pl.ds(o, SC_LANES)]
        t3 = tf_v[2, pl.ds(o, SC_LANES)]
        tsq = jnp.zeros((SC_LANES,), jnp.float32)
        for c in range(3):
            cc = jnp.full((SC_LANES,), c, jnp.int32)
            g = (plsc.load_gather(tv_v, [t1 * 3 + cc])
                 + plsc.load_gather(tv_v, [t2 * 3 + cc])
                 + plsc.load_gather(tv_v, [t3 * 3 + cc])) * third
            tb_v[c, pl.ds(u * SC_LANES, SC_LANES)] = g
            tsq = tsq + g * g
        tq_v[pl.ds(u * SC_LANES, SC_LANES)] = tsq
    for c in range(3):
        pltpu.sync_copy(tb_v.at[c], tb_h.at[pl.ds(c * N_TGT + tbase, t_per_w)])
    pltpu.sync_copy(tq_v, tq_h.at[pl.ds(tbase, t_per_w)])


def _sc_gather(svr, tvr, sft, tft, wst):
    n_sv = svr.shape[0]
    n_tv = tvr.shape[0]
    fn = pl.kernel(
        _sc_body,
        out_type=[
            jax.ShapeDtypeStruct((3 * F_SRC,), jnp.float32),
            jax.ShapeDtypeStruct((3 * NPF * F_SRC,), jnp.float32),
            jax.ShapeDtypeStruct((3 * N_TGT,), jnp.float32),
            jax.ShapeDtypeStruct((F_SRC,), jnp.float32),
            jax.ShapeDtypeStruct((N_PTS,), jnp.float32),
            jax.ShapeDtypeStruct((N_TGT,), jnp.float32),
        ],
        mesh=plsc.VectorSubcoreMesh(core_axis_name="c", subcore_axis_name="s"),
        compiler_params=pltpu.CompilerParams(needs_layout_passes=False),
        scratch_types=[
            pltpu.VMEM((n_sv * 3,), jnp.float32),
            pltpu.VMEM((n_tv * 3,), jnp.float32),
            pltpu.VMEM((3, F_SRC), jnp.int32),
            pltpu.VMEM((3, N_TGT), jnp.int32),
            pltpu.VMEM((3, NPF, F_SRC), jnp.float32),
            pltpu.VMEM((3, F_SRC // SC_WORKERS), jnp.float32),
            pltpu.VMEM((3, NPF, F_SRC // SC_WORKERS), jnp.float32),
            pltpu.VMEM((3, N_TGT // SC_WORKERS), jnp.float32),
            pltpu.VMEM((F_SRC // SC_WORKERS,), jnp.float32),
            pltpu.VMEM((NPF * F_SRC // SC_WORKERS,), jnp.float32),
            pltpu.VMEM((N_TGT // SC_WORKERS,), jnp.float32),
        ],
    )
    sb_f, pt_f, tb_f, sq_f, pq_f, tq_f = fn(
        svr.reshape(-1), tvr.reshape(-1), sft, tft, wst)
    return (sb_f.reshape(3, F_SRC), pt_f.reshape(3, NPF, F_SRC),
            tb_f.reshape(3, N_TGT), sq_f, pq_f, tq_f)


def _sample_weights():
    rk = jax.random.key(42)
    r1 = jnp.sqrt(jax.random.uniform(jax.random.fold_in(rk, 0), (F_SRC, NPF), dtype=jnp.float32))
    r2 = jax.random.uniform(jax.random.fold_in(rk, 1), (F_SRC, NPF), dtype=jnp.float32)
    w1 = 1.0 - r1
    w2 = r1 * (1.0 - r2)
    w3 = r1 * r2
    return jnp.stack([w1.T, w2.T, w3.T])


try:
    import numpy as _np
    _WST_CONST = _np.asarray(jax.device_get(_sample_weights()))
except Exception:
    _WST_CONST = None


def _wst():
    return _WST_CONST if _WST_CONST is not None else _sample_weights()


def kernel(source_vertices, source_faces, target_vertices, target_faces, face_probs):
    sv = source_vertices[0]
    tv = target_vertices[0]
    sf = source_faces.astype(jnp.int32)
    tf = target_faces.astype(jnp.int32)
    sb_t, pt_t, tb_t, sq_f, pq_f, tq_f = _sc_gather(sv, tv, sf.T, tf, _wst())
    points = jnp.transpose(pt_t, (2, 1, 0)).reshape(N_PTS, 3)

    pxb = jnp.zeros((N_PTS, 128), jnp.bfloat16).at[:, 0:3].set(
        points.astype(jnp.bfloat16))
    sbb = jnp.zeros((F_SRC, 128), jnp.bfloat16).at[:, 0:3].set(
        sb_t.T.astype(jnp.bfloat16))
    sxbt = jnp.zeros((128, F_SRC), jnp.bfloat16).at[0:3, :].set(
        sb_t.astype(jnp.bfloat16))
    tybt = jnp.zeros((128, N_TGT), jnp.bfloat16).at[0:3, :].set(
        tb_t.astype(jnp.bfloat16))
    px2c = pq_f[:, None]
    sx2r = sq_f[None, :]
    sb2c = sq_f[:, None]
    ty2r = tq_f[None, :]
    pr_row = face_probs[None, :]
    pc_col = face_probs[:, None]
    py_col = jnp.repeat(face_probs, NPF)[:, None]

    loss = _dense_loss(pxb, sbb, pc_col, py_col, px2c, sx2r, sb2c, ty2r,
                       sxbt, tybt, pr_row)
    return loss[0, 0]


# speedup vs baseline: 19.5762x; 1.0578x over previous
import jax
import jax.numpy as jnp
from jax import lax
from jax.experimental import pallas as pl
from jax.experimental.pallas import tpu as pltpu
from jax.experimental.pallas import tpu_sc as plsc

SC_CORES = 2
SC_SUBCORES = 16
SC_WORKERS = SC_CORES * SC_SUBCORES
SC_LANES = 16

NPF = 4
KNN = 5
F_SRC = 2048
N_PTS = F_SRC * NPF
N_TGT = 4096
BLK_P = 1024
BLK_F = BLK_P // NPF
GRID = N_PTS // BLK_P

IDX_BITS = 11
IDX_MASK = (1 << IDX_BITS) - 1


def _dense_body(pxb_ref, sbb_ref, pc_ref, py_ref, px2_ref, sx2_ref, sb2_ref,
                ty2_ref, sxbt_ref, tybt_ref, pr_ref, out_ref):
    b = pl.program_id(0)
    pc = pc_ref[...]
    py = py_ref[...]
    pr = pr_ref[...]
    px2 = px2_ref[...]
    sx2 = sx2_ref[...]
    sb2 = sb2_ref[...]
    ty2 = ty2_ref[...]

    cross = jnp.dot(pxb_ref[...], sxbt_ref[...],
                    preferred_element_type=jnp.float32)
    d = jnp.maximum(px2 + sx2 - 2.0 * cross, 0.0)

    ii = lax.broadcasted_iota(jnp.int32, (BLK_P, F_SRC), 1)
    key = ((d.view(jnp.int32) & ~IDX_MASK) | ii).view(jnp.float32)
    face_col = (lax.broadcasted_iota(jnp.int32, (BLK_P, 1), 0) + b * BLK_P) // NPF
    tot = jnp.zeros((BLK_P, 1), jnp.float32)
    s_self = jnp.zeros((BLK_P, 1), jnp.float32)
    has_self = jnp.zeros((BLK_P, 1), jnp.bool_)
    last_pd = jnp.zeros((BLK_P, 1), jnp.float32)
    inf = jnp.float32(jnp.inf)
    for i in range(KNN + 1):
        kmin = jnp.min(key, axis=1, keepdims=True)
        oh = key == kmin
        p = jnp.sum(jnp.where(oh, pr, 0.0), axis=1, keepdims=True)
        kbits = kmin.view(jnp.int32)
        dq = (kbits & ~IDX_MASK).view(jnp.float32)
        idx = kbits & IDX_MASK
        pd = p * dq
        tot = tot + pd
        selfhit = idx == face_col
        s_self = s_self + jnp.where(selfhit, pd, 0.0)
        has_self = has_self | selfhit
        if i == KNN:
            last_pd = pd
        else:
            key = jnp.where(oh, inf, key)
    mean_term = jnp.where(has_self, tot - s_self, tot - last_pd) * (1.0 / KNN)

    ct = jnp.dot(pxb_ref[...], tybt_ref[...],
                 preferred_element_type=jnp.float32)
    mt = jnp.maximum(
        px2 + jnp.min(ty2 - 2.0 * ct, axis=1, keepdims=True), 0.0)
    cf = jnp.dot(sbb_ref[...], tybt_ref[...],
                 preferred_element_type=jnp.float32)
    fmin = jnp.maximum(
        sb2 + jnp.min(ty2 - 2.0 * cf, axis=1, keepdims=True), 0.0)

    rev = jnp.sum(py * mt + (1.0 - py) * mean_term)
    fwd = jnp.sum(pc * fmin)

    @pl.when(b == 0)
    def _():
        out_ref[...] = jnp.zeros((1, 1), jnp.float32)

    out_ref[...] += (rev + fwd).reshape(1, 1)


def _dense_loss(pxb, sbb, pc_col, py_col, px2c, sx2r, sb2c, ty2r, sxbt,
                tybt, pr_row):
    return pl.pallas_call(
        _dense_body,
        grid=(GRID,),
        in_specs=[
            pl.BlockSpec((BLK_P, 128), lambda b: (b, 0)),
            pl.BlockSpec((BLK_F, 128), lambda b: (b, 0)),
            pl.BlockSpec((BLK_F, 1), lambda b: (b, 0)),
            pl.BlockSpec((BLK_P, 1), lambda b: (b, 0)),
            pl.BlockSpec((BLK_P, 1), lambda b: (b, 0)),
            pl.BlockSpec((1, F_SRC), lambda b: (0, 0)),
            pl.BlockSpec((BLK_F, 1), lambda b: (b, 0)),
            pl.BlockSpec((1, N_TGT), lambda b: (0, 0)),
            pl.BlockSpec((128, F_SRC), lambda b: (0, 0)),
            pl.BlockSpec((128, N_TGT), lambda b: (0, 0)),
            pl.BlockSpec((1, F_SRC), lambda b: (0, 0)),
        ],
        out_specs=pl.BlockSpec((1, 1), lambda b: (0, 0)),
        out_shape=jax.ShapeDtypeStruct((1, 1), jnp.float32),
    )(pxb, sbb, pc_col, py_col, px2c, sx2r, sb2c, ty2r, sxbt, tybt, pr_row)


def _sc_body(sv_h, tv_h, sf_h, tf_h, w_h, sb_h, pt_h, tb_h, sq_h, pq_h, tq_h,
             sv_v, tv_v, sf_v, tf_v, w_v, bb_v, pp_v, tb_v, bq_v, pq_v, tq_v):
    wid = lax.axis_index("s") * SC_CORES + lax.axis_index("c")
    pltpu.sync_copy(sv_h, sv_v)
    pltpu.sync_copy(tv_h, tv_v)
    pltpu.sync_copy(sf_h, sf_v)
    pltpu.sync_copy(tf_h, tf_v)
    pltpu.sync_copy(w_h, w_v)

    f_per_w = F_SRC // SC_WORKERS
    t_per_w = N_TGT // SC_WORKERS
    base = wid * f_per_w
    third = jnp.float32(1.0 / 3.0)
    lane4 = jnp.arange(SC_LANES, dtype=jnp.int32) * NPF
    for u in range(f_per_w // SC_LANES):
        o = base + u * SC_LANES
        f1 = sf_v[0, pl.ds(o, SC_LANES)]
        f2 = sf_v[1, pl.ds(o, SC_LANES)]
        f3 = sf_v[2, pl.ds(o, SC_LANES)]
        bsq = jnp.zeros((SC_LANES,), jnp.float32)
        psq = [jnp.zeros((SC_LANES,), jnp.float32) for _ in range(NPF)]
        for c in range(3):
            cc = jnp.full((SC_LANES,), c, jnp.int32)
            v1 = plsc.load_gather(sv_v, [f1 * 3 + cc])
            v2 = plsc.load_gather(sv_v, [f2 * 3 + cc])
            v3 = plsc.load_gather(sv_v, [f3 * 3 + cc])
            bary = (v1 + v2 + v3) * third
            bb_v[c, pl.ds(u * SC_LANES, SC_LANES)] = bary
            bsq = bsq + bary * bary
            for j in range(NPF):
                w1c = w_v[0, j, pl.ds(o, SC_LANES)]
                w2c = w_v[1, j, pl.ds(o, SC_LANES)]
                w3c = w_v[2, j, pl.ds(o, SC_LANES)]
                pt = w1c * v1 + w2c * v2 + w3c * v3
                pp_v[c, j, pl.ds(u * SC_LANES, SC_LANES)] = pt
                psq[j] = psq[j] + pt * pt
        bq_v[pl.ds(u * SC_LANES, SC_LANES)] = bsq
        for j in range(NPF):
            plsc.store_scatter(
                pq_v, [lane4 + (u * SC_LANES * NPF + j)], psq[j])
    for c in range(3):
        pltpu.sync_copy(bb_v.at[c], sb_h.at[pl.ds(c * F_SRC + base, f_per_w)])
        for j in range(NPF):
            pltpu.sync_copy(pp_v.at[c, j],
                            pt_h.at[pl.ds((c * NPF + j) * F_SRC + base, f_per_w)])
    pltpu.sync_copy(bq_v, sq_h.at[pl.ds(base, f_per_w)])
    pltpu.sync_copy(pq_v, pq_h.at[pl.ds(base * NPF, f_per_w * NPF)])

    tbase = wid * t_per_w
    for u in range(t_per_w // SC_LANES):
        o = tbase + u * SC_LANES
        t1 = tf_v[0, pl.ds(o, SC_LANES)]
        t2 = tf_v[1, pl.ds(o, SC_LANES)]
        t3 = tf_v[2, pl.ds(o, SC_LANES)]
        tsq = jnp.zeros((SC_LANES,), jnp.float32)
        for c in range(3):
            cc = jnp.full((SC_LANES,), c, jnp.int32)
            g = (plsc.load_gather(tv_v, [t1 * 3 + cc])
                 + plsc.load_gather(tv_v, [t2 * 3 + cc])
                 + plsc.load_gather(tv_v, [t3 * 3 + cc])) * third
            tb_v[c, pl.ds(u * SC_LANES, SC_LANES)] = g
            tsq = tsq + g * g
        tq_v[pl.ds(u * SC_LANES, SC_LANES)] = tsq
    for c in range(3):
        pltpu.sync_copy(tb_v.at[c], tb_h.at[pl.ds(c * N_TGT + tbase, t_per_w)])
    pltpu.sync_copy(tq_v, tq_h.at[pl.ds(tbase, t_per_w)])


def _sc_gather(svr, tvr, sft, tft, wst):
    n_sv = svr.shape[0]
    n_tv = tvr.shape[0]
    fn = pl.kernel(
        _sc_body,
        out_type=[
            jax.ShapeDtypeStruct((3 * F_SRC,), jnp.float32),
            jax.ShapeDtypeStruct((3 * NPF * F_SRC,), jnp.float32),
            jax.ShapeDtypeStruct((3 * N_TGT,), jnp.float32),
            jax.ShapeDtypeStruct((F_SRC,), jnp.float32),
            jax.ShapeDtypeStruct((N_PTS,), jnp.float32),
            jax.ShapeDtypeStruct((N_TGT,), jnp.float32),
        ],
        mesh=plsc.VectorSubcoreMesh(core_axis_name="c", subcore_axis_name="s"),
        compiler_params=pltpu.CompilerParams(needs_layout_passes=False),
        scratch_types=[
            pltpu.VMEM((n_sv * 3,), jnp.float32),
            pltpu.VMEM((n_tv * 3,), jnp.float32),
            pltpu.VMEM((3, F_SRC), jnp.int32),
            pltpu.VMEM((3, N_TGT), jnp.int32),
            pltpu.VMEM((3, NPF, F_SRC), jnp.float32),
            pltpu.VMEM((3, F_SRC // SC_WORKERS), jnp.float32),
            pltpu.VMEM((3, NPF, F_SRC // SC_WORKERS), jnp.float32),
            pltpu.VMEM((3, N_TGT // SC_WORKERS), jnp.float32),
            pltpu.VMEM((F_SRC // SC_WORKERS,), jnp.float32),
            pltpu.VMEM((NPF * F_SRC // SC_WORKERS,), jnp.float32),
            pltpu.VMEM((N_TGT // SC_WORKERS,), jnp.float32),
        ],
    )
    sb_f, pt_f, tb_f, sq_f, pq_f, tq_f = fn(
        svr.reshape(-1), tvr.reshape(-1), sft, tft, wst)
    return (sb_f.reshape(3, F_SRC), pt_f.reshape(3, NPF, F_SRC),
            tb_f.reshape(3, N_TGT), sq_f, pq_f, tq_f)


def _sample_weights():
    rk = jax.random.key(42)
    r1 = jnp.sqrt(jax.random.uniform(jax.random.fold_in(rk, 0), (F_SRC, NPF), dtype=jnp.float32))
    r2 = jax.random.uniform(jax.random.fold_in(rk, 1), (F_SRC, NPF), dtype=jnp.float32)
    w1 = 1.0 - r1
    w2 = r1 * (1.0 - r2)
    w3 = r1 * r2
    return jnp.stack([w1.T, w2.T, w3.T])


try:
    import numpy as _np
    _WST_CONST = _np.asarray(jax.device_get(_sample_weights()))
except Exception:
    _WST_CONST = None


def _wst():
    return _WST_CONST if _WST_CONST is not None else _sample_weights()


def kernel(source_vertices, source_faces, target_vertices, target_faces, face_probs):
    sv = source_vertices[0]
    tv = target_vertices[0]
    sf = source_faces.astype(jnp.int32)
    tf = target_faces.astype(jnp.int32)
    sb_t, pt_t, tb_t, sq_f, pq_f, tq_f = _sc_gather(sv, tv, sf.T, tf, _wst())
    points = jnp.transpose(pt_t, (2, 1, 0)).reshape(N_PTS, 3)

    pxb = jnp.zeros((N_PTS, 128), jnp.bfloat16).at[:, 0:3].set(
        points.astype(jnp.bfloat16))
    sbb = jnp.zeros((F_SRC, 128), jnp.bfloat16).at[:, 0:3].set(
        sb_t.T.astype(jnp.bfloat16))
    sxbt = jnp.zeros((128, F_SRC), jnp.bfloat16).at[0:3, :].set(
        sb_t.astype(jnp.bfloat16))
    tybt = jnp.zeros((128, N_TGT), jnp.bfloat16).at[0:3, :].set(
        tb_t.astype(jnp.bfloat16))
    px2c = pq_f[:, None]
    sx2r = sq_f[None, :]
    sb2c = sq_f[:, None]
    ty2r = tq_f[None, :]
    pr_row = face_probs[None, :]
    pc_col = face_probs[:, None]
    py_col = jnp.repeat(face_probs, NPF)[:, None]

    loss = _dense_loss(pxb, sbb, pc_col, py_col, px2c, sx2r, sb2c, ty2r,
                       sxbt, tybt, pr_row)
    return loss[0, 0]
